# baseline (speedup 1.0000x reference)
import jax
import jax.numpy as jnp
from jax import lax
from jax.experimental import pallas as pl
from jax.experimental.pallas import tpu as pltpu
from jax.experimental.pallas import tpu_sc as plsc

MU = 0.5
TEMP = 0.1
B = 4096
D = 64
NPAIR = 8192
EPS = 1e-12

NC = 2
NS = 16
NL = 16
NW = NC * NS
CHUNK = 128

f32 = jnp.float32
i32 = jnp.int32


D2 = 2 * D


def _sc_gather_body(uid_h, iid_h, eid_h, p0_h, p1_h, ut_h, it_h, et_h,
                    u_rows_h, i_rows_h, e_rows_h, u2_h, i2_h, e2_h,
                    cu_h, ci_h, ce_h,
                    idx_v, idx2_v, idx2s_v, ids_v, rows_v, sem):
    wid = lax.axis_index("s") * NC + lax.axis_index("c")

    def shift_inplace(dst_v, src_v):
        for g in range(CHUNK // NL):
            dst_v[pl.ds(g * NL, NL)] = src_v[pl.ds(g * NL, NL)] >> 1

    def direct(ids_hbm, table_hbm, out_hbm):
        base = wid * (B // NW)
        pltpu.sync_copy(ids_hbm.at[pl.ds(base, CHUNK)], idx_v)
        shift_inplace(idx2s_v, idx_v)
        pltpu.async_copy(table_hbm.at[idx2s_v], rows_v, sem).wait()
        pltpu.sync_copy(rows_v, out_hbm.at[pl.ds(base, CHUNK)])

    def composed(p_hbm, ids_hbm, table_hbm, out_hbm, cidx_hbm):
        pltpu.sync_copy(ids_hbm, ids_v)
        for c in range(NPAIR // NW // CHUNK):
            base = wid * (NPAIR // NW) + c * CHUNK
            pltpu.sync_copy(p_hbm.at[pl.ds(base, CHUNK)], idx_v)
            for g in range(CHUNK // NL):
                i16 = idx_v[pl.ds(g * NL, NL)]
                comp = plsc.load_gather(ids_v, [i16])
                idx2_v[pl.ds(g * NL, NL)] = comp
                idx2s_v[pl.ds(g * NL, NL)] = comp >> 1
            pltpu.async_copy(table_hbm.at[idx2s_v], rows_v, sem).wait()
            pltpu.sync_copy(rows_v, out_hbm.at[pl.ds(base, CHUNK)])
            pltpu.sync_copy(idx2_v, cidx_hbm.at[pl.ds(base, CHUNK)])

    direct(uid_h, ut_h, u_rows_h)
    direct(iid_h, it_h, i_rows_h)
    direct(eid_h, et_h, e_rows_h)
    composed(p0_h, uid_h, ut_h, u2_h, cu_h)
    composed(p0_h, iid_h, it_h, i2_h, ci_h)
    composed(p1_h, eid_h, et_h, e2_h, ce_h)


def _sc_gather(uid, iid, eid, p0, p1, ut, it, et):
    mesh = plsc.VectorSubcoreMesh(core_axis_name="c", subcore_axis_name="s",
                                  num_cores=NC, num_subcores=NS)
    out_type = (
        jax.ShapeDtypeStruct((B, D2), f32),
        jax.ShapeDtypeStruct((B, D2), f32),
        jax.ShapeDtypeStruct((B, D2), f32),
        jax.ShapeDtypeStruct((NPAIR, D2), f32),
        jax.ShapeDtypeStruct((NPAIR, D2), f32),
        jax.ShapeDtypeStruct((NPAIR, D2), f32),
        jax.ShapeDtypeStruct((NPAIR,), i32),
        jax.ShapeDtypeStruct((NPAIR,), i32),
        jax.ShapeDtypeStruct((NPAIR,), i32),
    )
    scratch_types = [
        pltpu.VMEM((CHUNK,), i32),
        pltpu.VMEM((CHUNK,), i32),
        pltpu.VMEM((CHUNK,), i32),
        pltpu.VMEM((B,), i32),
        pltpu.VMEM((CHUNK, D2), f32),
        pltpu.SemaphoreType.DMA,
    ]
    fn = pl.kernel(_sc_gather_body, out_type=out_type, mesh=mesh,
                   scratch_types=scratch_types,
                   compiler_params=pltpu.CompilerParams(
                       needs_layout_passes=False,
                       use_tc_tiling_on_sc=False))
    return fn(uid, iid, eid, p0, p1, ut, it, et)


def _halfsel(packed, ids):
    par = (ids & 1) == 1
    return jnp.where(par, packed[:, D:], packed[:, :D])


def _prep_body(u_ref, i_ref, e_ref, uid_ref, iid_ref, eid_ref,
               u2_ref, i2_ref, e2_ref, cu_ref, ci_ref, ce_ref,
               ui_ref, en_ref, zd_ref, zp_ref):
    u = _halfsel(u_ref[...], uid_ref[...])
    it = _halfsel(i_ref[...], iid_ref[...])
    ex = _halfsel(e_ref[...], eid_ref[...])
    mix = MU * u + (1.0 - MU) * it
    nm = jnp.sqrt(jnp.sum(mix * mix, axis=1, keepdims=True))
    ui = mix / jnp.maximum(nm, EPS)
    ne = jnp.sqrt(jnp.sum(ex * ex, axis=1, keepdims=True))
    en = ex / jnp.maximum(ne, EPS)
    ui_ref[...] = ui
    en_ref[...] = en
    zd_ref[...] = jnp.sum(ui * en, axis=1, keepdims=True) / TEMP

    u2 = _halfsel(u2_ref[...], cu_ref[...])
    i2 = _halfsel(i2_ref[...], ci_ref[...])
    e2 = _halfsel(e2_ref[...], ce_ref[...])
    mix2 = MU * u2 + (1.0 - MU) * i2
    n1 = jnp.maximum(jnp.sqrt(jnp.sum(mix2 * mix2, axis=1, keepdims=True)), EPS)
    n2 = jnp.maximum(jnp.sqrt(jnp.sum(e2 * e2, axis=1, keepdims=True)), EPS)
    dt = jnp.sum(mix2 * e2, axis=1, keepdims=True)
    zp_ref[...] = dt / (n1 * n2) / TEMP


def _tc_prep(u_rows, i_rows, e_rows, uid, iid, eid, u2, i2, e2, cu, ci, ce):
    G = 8
    RB = B // G
    PB = NPAIR // G
    return pl.pallas_call(
        _prep_body,
        grid=(G,),
        in_specs=[
            pl.BlockSpec((RB, D2), lambda g: (g, 0)),
            pl.BlockSpec((RB, D2), lambda g: (g, 0)),
            pl.BlockSpec((RB, D2), lambda g: (g, 0)),
            pl.BlockSpec((RB, 1), lambda g: (g, 0)),
            pl.BlockSpec((RB, 1), lambda g: (g, 0)),
            pl.BlockSpec((RB, 1), lambda g: (g, 0)),
            pl.BlockSpec((PB, D2), lambda g: (g, 0)),
            pl.BlockSpec((PB, D2), lambda g: (g, 0)),
            pl.BlockSpec((PB, D2), lambda g: (g, 0)),
            pl.BlockSpec((PB, 1), lambda g: (g, 0)),
            pl.BlockSpec((PB, 1), lambda g: (g, 0)),
            pl.BlockSpec((PB, 1), lambda g: (g, 0)),
        ],
        out_specs=[
            pl.BlockSpec((RB, D), lambda g: (g, 0)),
            pl.BlockSpec((RB, D), lambda g: (g, 0)),
            pl.BlockSpec((RB, 1), lambda g: (g, 0)),
            pl.BlockSpec((PB, 1), lambda g: (g, 0)),
        ],
        out_shape=[
            jax.ShapeDtypeStruct((B, D), f32),
            jax.ShapeDtypeStruct((B, D), f32),
            jax.ShapeDtypeStruct((B, 1), f32),
            jax.ShapeDtypeStruct((NPAIR, 1), f32),
        ],
    )(u_rows, i_rows, e_rows, uid, iid, eid, u2, i2, e2, cu, ci, ce)


_ZT = 512


def _zpass_body(ui_ref, en_ref, rse_ref, rsz_ref, cse_ref):
    i = pl.program_id(0)
    j = pl.program_id(1)
    zt = lax.dot_general(ui_ref[...].astype(jnp.bfloat16),
                         en_ref[...].astype(jnp.bfloat16),
                         (((1,), (1,)), ((), ())),
                         preferred_element_type=f32) * (1.0 / TEMP)
    ez = jnp.exp(zt)
    rse_part = jnp.sum(ez, axis=1, keepdims=True)
    rsz_part = jnp.sum(zt, axis=1, keepdims=True)
    cse_part = jnp.sum(ez, axis=0, keepdims=True)

    @pl.when(j == 0)
    def _():
        rse_ref[...] = rse_part
        rsz_ref[...] = rsz_part

    @pl.when(j != 0)
    def _():
        rse_ref[...] += rse_part
        rsz_ref[...] += rsz_part

    @pl.when(i == 0)
    def _():
        cse_ref[:, pl.ds(j * _ZT, _ZT)] = cse_part

    @pl.when(i != 0)
    def _():
        cse_ref[:, pl.ds(j * _ZT, _ZT)] += cse_part


def _tc_zpass(ui_n, e_n):
    G = B // _ZT
    return pl.pallas_call(
        _zpass_body,
        grid=(G, G),
        in_specs=[
            pl.BlockSpec((_ZT, D), lambda i, j: (i, 0)),
            pl.BlockSpec((_ZT, D), lambda i, j: (j, 0)),
        ],
        out_specs=[
            pl.BlockSpec((_ZT, 1), lambda i, j: (i, 0)),
            pl.BlockSpec((_ZT, 1), lambda i, j: (i, 0)),
            pl.BlockSpec((1, B), lambda i, j: (0, 0)),
        ],
        out_shape=[
            jax.ShapeDtypeStruct((B, 1), f32),
            jax.ShapeDtypeStruct((B, 1), f32),
            jax.ShapeDtypeStruct((1, B), f32),
        ],
        compiler_params=pltpu.CompilerParams(
            dimension_semantics=("arbitrary", "arbitrary")),
    )(ui_n, e_n)


_PT = 1024
_CT = 512


def _pairs_body(p0c_ref, p1c_ref, zp_ref, p0r_ref, p1r_ref, cseg_ref,
                cset_ref, cnt_ref, lsep_ref, lsec_ref, w3_ref):
    j = pl.program_id(1)
    nj = pl.num_programs(1)

    p0c = p0c_ref[...]
    p1c = p1c_ref[...]
    keyc = p0c * B + p1c
    keyr = p0r_ref[...] * B + p1r_ref[...]
    eq = jnp.where(keyc == keyr, 1.0, 0.0)
    cnt_part = jnp.sum(eq, axis=1, keepdims=True)

    @pl.when(j == 0)
    def _():
        cnt_ref[...] = cnt_part + jnp.where(p0c == p1c, 1.0, 0.0)

    @pl.when(j != 0)
    def _():
        cnt_ref[...] += cnt_part

    def g_part():
        lse_g = jnp.log(cseg_ref[...])
        cols_g = lax.broadcasted_iota(i32, (1, _PT), 1) + j * _PT
        return jnp.sum(jnp.where(p1c == cols_g, lse_g, 0.0),
                       axis=1, keepdims=True)

    @pl.when(j == 0)
    def _():
        lsep_ref[...] = g_part()

    @pl.when(jnp.logical_and(j != 0, j < B // _PT))
    def _():
        lsep_ref[...] += g_part()

    i = pl.program_id(0)
    lse_t = jnp.log(cset_ref[...])
    rows_t = lax.broadcasted_iota(i32, (_CT, 1), 0) + i * _CT
    cols_t = lax.broadcasted_iota(i32, (1, _CT), 1) + j * _CT
    t_part = jnp.sum(jnp.where(rows_t == cols_t, lse_t, 0.0),
                     axis=1, keepdims=True)

    @pl.when(j == 0)
    def _():
        lsec_ref[...] = t_part

    @pl.when(j != 0)
    def _():
        lsec_ref[...] += t_part

    @pl.when(j == nj - 1)
    def _():
        w = 1.0 / cnt_ref[...]
        v = w * (2.0 * zp_ref[...] - lsep_ref[...])
        d = jnp.where(p0c == p1c, 1.0, 0.0)
        w3_ref[...] = jnp.concatenate([w, v, d, jnp.zeros_like(w)], axis=1)


def _tc_pairs(p0c, p1c, p0r, p1r, z_p, c_se):
    GI = NPAIR // _PT
    GJ = NPAIR // _PT
    return pl.pallas_call(
        _pairs_body,
        grid=(GI, GJ),
        in_specs=[
            pl.BlockSpec((_PT, 1), lambda i, j: (i, 0)),
            pl.BlockSpec((_PT, 1), lambda i, j: (i, 0)),
            pl.BlockSpec((_PT, 1), lambda i, j: (i, 0)),
            pl.BlockSpec((1, _PT), lambda i, j: (0, j)),
            pl.BlockSpec((1, _PT), lambda i, j: (0, j)),
            pl.BlockSpec((1, _PT), lambda i, j: (0, jnp.minimum(j, B // _PT - 1))),
            pl.BlockSpec((1, _CT), lambda i, j: (0, j)),
        ],
        out_specs=[
            pl.BlockSpec((_PT, 1), lambda i, j: (i, 0)),
            pl.BlockSpec((_PT, 1), lambda i, j: (i, 0)),
            pl.BlockSpec((_CT, 1), lambda i, j: (i, 0)),
            pl.BlockSpec((_PT, 4), lambda i, j: (i, 0)),
        ],
        out_shape=[
            jax.ShapeDtypeStruct((NPAIR, 1), f32),
            jax.ShapeDtypeStruct((NPAIR, 1), f32),
            jax.ShapeDtypeStruct((B, 1), f32),
            jax.ShapeDtypeStruct((NPAIR, 4), f32),
        ],
        compiler_params=pltpu.CompilerParams(
            dimension_semantics=("arbitrary", "arbitrary")),
    )(p0c, p1c, z_p, p0r, p1r, c_se, c_se)


_AT = 1024


def _agg_body(p0r_ref, w3_ref, agg_ref):
    t = pl.program_id(0)
    k = pl.program_id(1)
    bf16 = jnp.bfloat16
    rows = lax.broadcasted_iota(i32, (_AT, 1), 0) + t * _AT
    oht = jnp.where(rows == p0r_ref[...], 1.0, 0.0).astype(bf16)
    part = lax.dot_general(oht, w3_ref[...].astype(bf16),
                           (((1,), (0,)), ((), ())),
                           preferred_element_type=f32)

    @pl.when(k == 0)
    def _():
        agg_ref[...] = part

    @pl.when(k != 0)
    def _():
        agg_ref[...] += part


def _tc_agg(p0r, w3):
    GT = B // _AT
    GK = NPAIR // _PT
    return pl.pallas_call(
        _agg_body,
        grid=(GT, GK),
        in_specs=[
            pl.BlockSpec((1, _PT), lambda t, k: (0, k)),
            pl.BlockSpec((_PT, 4), lambda t, k: (k, 0)),
        ],
        out_specs=pl.BlockSpec((_AT, 4), lambda t, k: (t, 0)),
        out_shape=jax.ShapeDtypeStruct((B, 4), f32),
        compiler_params=pltpu.CompilerParams(
            dimension_semantics=("arbitrary", "arbitrary")),
    )(p0r, w3)


def _final_body(a_ref, bv_ref, m_ref, rse_ref, rsz_ref, zd_ref, lsec_ref,
                out_ref):
    a = a_ref[...]
    bv = bv_ref[...]
    m = m_ref[...]
    lse_row = jnp.log(rse_ref[...])
    rs_z = rsz_ref[...]
    z_diag = zd_ref[...]
    lse_col = lsec_ref[...]
    s_col = jnp.sum(lse_col)

    w_d = 1.0 / (1.0 + m)
    num_pos = a + w_d
    loss_pos = bv - a * lse_row + w_d * (2.0 * z_diag - lse_row - lse_col)
    rowsum_ls = 2.0 * rs_z - float(B) * lse_row - s_col
    loss_neg = rowsum_ls - loss_pos
    num_neg = float(B) - num_pos
    loss = -jnp.sum(loss_pos / num_pos + loss_neg / num_neg) / float(B)
    out_ref[...] = loss * jnp.ones((1, 1), f32)


def _tc_final(agg, r_se, rs_z, z_diag, lse_col_c):
    lane = (32, 128)
    args = [
        agg[:, 0:1].reshape(lane), agg[:, 1:2].reshape(lane),
        agg[:, 2:3].reshape(lane), r_se.reshape(lane), rs_z.reshape(lane),
        z_diag.reshape(lane), lse_col_c.reshape(lane),
    ]
    return pl.pallas_call(
        _final_body,
        out_shape=jax.ShapeDtypeStruct((1, 1), f32),
    )(*args)


def kernel(user_ids, item_ids, exp_ids, pos_indices, user_table, item_table,
           exp_table):
    uid = user_ids.astype(i32)
    iid = item_ids.astype(i32)
    eid = exp_ids.astype(i32)
    p0 = pos_indices[:, 0].astype(i32)
    p1 = pos_indices[:, 1].astype(i32)

    utp = user_table.reshape(user_table.shape[0] // 2, D2)
    itp = item_table.reshape(item_table.shape[0] // 2, D2)
    etp = exp_table.reshape(exp_table.shape[0] // 2, D2)
    (u_rows, i_rows, e_rows, u2, i2, e2, cu, ci, ce) = _sc_gather(
        uid, iid, eid, p0, p1, utp, itp, etp)

    ui_n, e_n, z_diag, z_p = _tc_prep(
        u_rows, i_rows, e_rows, uid.reshape(B, 1), iid.reshape(B, 1),
        eid.reshape(B, 1), u2, i2, e2, cu.reshape(NPAIR, 1),
        ci.reshape(NPAIR, 1), ce.reshape(NPAIR, 1))
    r_se, rs_z, c_se = _tc_zpass(ui_n, e_n)

    p0c = p0.reshape(NPAIR, 1)
    p1c = p1.reshape(NPAIR, 1)
    p0r = p0.reshape(1, NPAIR)
    p1r = p1.reshape(1, NPAIR)
    cnt, lse_p, lse_col_c, w3 = _tc_pairs(p0c, p1c, p0r, p1r, z_p, c_se)
    agg = _tc_agg(p0r, w3)
    out = _tc_final(agg, r_se, rs_z, z_diag, lse_col_c)
    return out[0, 0]


# speedup vs baseline: 2.1447x; 2.1447x over previous
import jax
import jax.numpy as jnp
from jax import lax
from jax.experimental import pallas as pl
from jax.experimental.pallas import tpu as pltpu
from jax.experimental.pallas import tpu_sc as plsc

MU = 0.5
TEMP = 0.1
B = 4096
D = 64
NPAIR = 8192
EPS = 1e-12

NC = 2
NS = 16
NL = 16
NW = NC * NS
CHUNK = 128

f32 = jnp.float32
i32 = jnp.int32


D2 = 2 * D


def _sc_mesh():
    return plsc.VectorSubcoreMesh(core_axis_name="c", subcore_axis_name="s",
                                  num_cores=NC, num_subcores=NS)


def _sc_stripe_body(uid_h, iid_h, eid_h, tu_h, ti_h, te_h,
                    pu_h, pi_h, pe_h, idx_v, tile_v, pack_v, sem):
    wid = lax.axis_index("s") * NC + lax.axis_index("c")
    base = wid * (B // NW)
    lanes = lax.broadcasted_iota(i32, (NL,), 0)
    for ids_h, t_h, o_h in ((uid_h, tu_h, pu_h), (iid_h, ti_h, pi_h),
                            (eid_h, te_h, pe_h)):
        pltpu.sync_copy(ids_h.at[pl.ds(base, CHUNK)], idx_v.at[pl.ds(0, CHUNK)])

        def row(k, _):
            i = idx_v[pl.ds(k, NL)][0]
            col0 = pl.multiple_of((i >> 7) << 7, D2)
            pltpu.async_copy(t_h.at[:, pl.ds(col0, D2)], tile_v, sem).wait()
            ii = jnp.broadcast_to(i - col0, (NL,))
            p = k >> 1
            off = (k & 1) * D
            for g in range(D // NL):
                vals = plsc.load_gather(tile_v, [lanes + g * NL, ii])
                pack_v[p, pl.ds(off + g * NL, NL)] = vals
            return 0

        lax.fori_loop(0, CHUNK, row, 0)
        pltpu.sync_copy(pack_v, o_h.at[pl.ds(wid * (CHUNK // 2), CHUNK // 2)])


def _sc_stripe(uid, iid, eid, tu, ti, te):
    out_type = (
        jax.ShapeDtypeStruct((B // 2, D2), f32),
        jax.ShapeDtypeStruct((B // 2, D2), f32),
        jax.ShapeDtypeStruct((B // 2, D2), f32),
    )
    scratch_types = [
        pltpu.VMEM((CHUNK + NL,), i32),
        pltpu.VMEM((D, D2), f32),
        pltpu.VMEM((CHUNK // 2, D2), f32),
        pltpu.SemaphoreType.DMA,
    ]
    fn = pl.kernel(_sc_stripe_body, out_type=out_type, mesh=_sc_mesh(),
                   scratch_types=scratch_types,
                   compiler_params=pltpu.CompilerParams(
                       needs_layout_passes=False,
                       use_tc_tiling_on_sc=True))
    return fn(uid, iid, eid, tu, ti, te)


def _sc_pair_body(p0_h, p1_h, pu_h, pi_h, pe_h, u2_h, i2_h, e2_h,
                  idx_v, idxs_v, rows_v, sem):
    wid = lax.axis_index("s") * NC + lax.axis_index("c")
    for p_h, src_h, out_h in ((p0_h, pu_h, u2_h), (p0_h, pi_h, i2_h),
                              (p1_h, pe_h, e2_h)):
        for c in range(NPAIR // NW // CHUNK):
            base = wid * (NPAIR // NW) + c * CHUNK
            pltpu.sync_copy(p_h.at[pl.ds(base, CHUNK)], idx_v)
            for g in range(CHUNK // NL):
                idxs_v[pl.ds(g * NL, NL)] = idx_v[pl.ds(g * NL, NL)] >> 1
            pltpu.async_copy(src_h.at[idxs_v], rows_v, sem).wait()
            pltpu.sync_copy(rows_v, out_h.at[pl.ds(base, CHUNK)])


def _sc_pair(p0, p1, pu, pi_, pe):
    out_type = (
        jax.ShapeDtypeStruct((NPAIR, D2), f32),
        jax.ShapeDtypeStruct((NPAIR, D2), f32),
        jax.ShapeDtypeStruct((NPAIR, D2), f32),
    )
    scratch_types = [
        pltpu.VMEM((CHUNK,), i32),
        pltpu.VMEM((CHUNK,), i32),
        pltpu.VMEM((CHUNK, D2), f32),
        pltpu.SemaphoreType.DMA,
    ]
    fn = pl.kernel(_sc_pair_body, out_type=out_type, mesh=_sc_mesh(),
                   scratch_types=scratch_types,
                   compiler_params=pltpu.CompilerParams(
                       needs_layout_passes=False,
                       use_tc_tiling_on_sc=True))
    return fn(p0, p1, pu, pi_, pe)


def _halfsel(packed, par):
    return jnp.where((par & 1) == 1, packed[:, D:], packed[:, :D])


def _prep_body(u_ref, i_ref, e_ref, u2_ref, i2_ref, e2_ref,
               p0c_ref, p1c_ref, ui_ref, en_ref, zd_ref, zp_ref):
    mix = MU * u_ref[...] + (1.0 - MU) * i_ref[...]
    nm = jnp.sqrt(jnp.sum(mix * mix, axis=1, keepdims=True))
    ui = mix / jnp.maximum(nm, EPS)
    ex = e_ref[...]
    ne = jnp.sqrt(jnp.sum(ex * ex, axis=1, keepdims=True))
    en = ex / jnp.maximum(ne, EPS)
    ui_ref[...] = ui
    en_ref[...] = en
    zd_ref[...] = jnp.sum(ui * en, axis=1, keepdims=True) / TEMP

    u2 = _halfsel(u2_ref[...], p0c_ref[...])
    i2 = _halfsel(i2_ref[...], p0c_ref[...])
    e2 = _halfsel(e2_ref[...], p1c_ref[...])
    mix2 = MU * u2 + (1.0 - MU) * i2
    n1 = jnp.maximum(jnp.sqrt(jnp.sum(mix2 * mix2, axis=1, keepdims=True)), EPS)
    n2 = jnp.maximum(jnp.sqrt(jnp.sum(e2 * e2, axis=1, keepdims=True)), EPS)
    dt = jnp.sum(mix2 * e2, axis=1, keepdims=True)
    zp_ref[...] = dt / (n1 * n2) / TEMP


def _tc_prep(u_rows, i_rows, e_rows, u2, i2, e2, p0c, p1c):
    G = 8
    RB = B // G
    PB = NPAIR // G
    return pl.pallas_call(
        _prep_body,
        grid=(G,),
        in_specs=[
            pl.BlockSpec((RB, D), lambda g: (g, 0)),
            pl.BlockSpec((RB, D), lambda g: (g, 0)),
            pl.BlockSpec((RB, D), lambda g: (g, 0)),
            pl.BlockSpec((PB, D2), lambda g: (g, 0)),
            pl.BlockSpec((PB, D2), lambda g: (g, 0)),
            pl.BlockSpec((PB, D2), lambda g: (g, 0)),
            pl.BlockSpec((PB, 1), lambda g: (g, 0)),
            pl.BlockSpec((PB, 1), lambda g: (g, 0)),
        ],
        out_specs=[
            pl.BlockSpec((RB, D), lambda g: (g, 0)),
            pl.BlockSpec((RB, D), lambda g: (g, 0)),
            pl.BlockSpec((RB, 1), lambda g: (g, 0)),
            pl.BlockSpec((PB, 1), lambda g: (g, 0)),
        ],
        out_shape=[
            jax.ShapeDtypeStruct((B, D), f32),
            jax.ShapeDtypeStruct((B, D), f32),
            jax.ShapeDtypeStruct((B, 1), f32),
            jax.ShapeDtypeStruct((NPAIR, 1), f32),
        ],
    )(u_rows, i_rows, e_rows, u2, i2, e2, p0c, p1c)


_ZT = 512


def _zpass_body(ui_ref, en_ref, rse_ref, rsz_ref, cse_ref):
    i = pl.program_id(0)
    j = pl.program_id(1)
    zt = lax.dot_general(ui_ref[...].astype(jnp.bfloat16),
                         en_ref[...].astype(jnp.bfloat16),
                         (((1,), (1,)), ((), ())),
                         preferred_element_type=f32) * (1.0 / TEMP)
    ez = jnp.exp(zt)
    rse_part = jnp.sum(ez, axis=1, keepdims=True)
    rsz_part = jnp.sum(zt, axis=1, keepdims=True)
    cse_part = jnp.sum(ez, axis=0, keepdims=True)

    @pl.when(j == 0)
    def _():
        rse_ref[...] = rse_part
        rsz_ref[...] = rsz_part

    @pl.when(j != 0)
    def _():
        rse_ref[...] += rse_part
        rsz_ref[...] += rsz_part

    @pl.when(i == 0)
    def _():
        cse_ref[:, pl.ds(j * _ZT, _ZT)] = cse_part

    @pl.when(i != 0)
    def _():
        cse_ref[:, pl.ds(j * _ZT, _ZT)] += cse_part


def _tc_zpass(ui_n, e_n):
    G = B // _ZT
    return pl.pallas_call(
        _zpass_body,
        grid=(G, G),
        in_specs=[
            pl.BlockSpec((_ZT, D), lambda i, j: (i, 0)),
            pl.BlockSpec((_ZT, D), lambda i, j: (j, 0)),
        ],
        out_specs=[
            pl.BlockSpec((_ZT, 1), lambda i, j: (i, 0)),
            pl.BlockSpec((_ZT, 1), lambda i, j: (i, 0)),
            pl.BlockSpec((1, B), lambda i, j: (0, 0)),
        ],
        out_shape=[
            jax.ShapeDtypeStruct((B, 1), f32),
            jax.ShapeDtypeStruct((B, 1), f32),
            jax.ShapeDtypeStruct((1, B), f32),
        ],
        compiler_params=pltpu.CompilerParams(
            dimension_semantics=("arbitrary", "arbitrary")),
    )(ui_n, e_n)


_PT = 1024
_CT = 512


def _pairs_body(p0c_ref, p1c_ref, zp_ref, p0r_ref, p1r_ref, cseg_ref,
                cset_ref, cnt_ref, lsep_ref, lsec_ref, w3_ref):
    j = pl.program_id(1)
    nj = pl.num_programs(1)

    p0c = p0c_ref[...]
    p1c = p1c_ref[...]
    keyc = p0c * B + p1c
    keyr = p0r_ref[...] * B + p1r_ref[...]
    eq = jnp.where(keyc == keyr, 1.0, 0.0)
    cnt_part = jnp.sum(eq, axis=1, keepdims=True)

    @pl.when(j == 0)
    def _():
        cnt_ref[...] = cnt_part + jnp.where(p0c == p1c, 1.0, 0.0)

    @pl.when(j != 0)
    def _():
        cnt_ref[...] += cnt_part

    def g_part():
        lse_g = jnp.log(cseg_ref[...])
        cols_g = lax.broadcasted_iota(i32, (1, _PT), 1) + j * _PT
        return jnp.sum(jnp.where(p1c == cols_g, lse_g, 0.0),
                       axis=1, keepdims=True)

    @pl.when(j == 0)
    def _():
        lsep_ref[...] = g_part()

    @pl.when(jnp.logical_and(j != 0, j < B // _PT))
    def _():
        lsep_ref[...] += g_part()

    i = pl.program_id(0)
    lse_t = jnp.log(cset_ref[...])
    rows_t = lax.broadcasted_iota(i32, (_CT, 1), 0) + i * _CT
    cols_t = lax.broadcasted_iota(i32, (1, _CT), 1) + j * _CT
    t_part = jnp.sum(jnp.where(rows_t == cols_t, lse_t, 0.0),
                     axis=1, keepdims=True)

    @pl.when(j == 0)
    def _():
        lsec_ref[...] = t_part

    @pl.when(j != 0)
    def _():
        lsec_ref[...] += t_part

    @pl.when(j == nj - 1)
    def _():
        w = 1.0 / cnt_ref[...]
        v = w * (2.0 * zp_ref[...] - lsep_ref[...])
        d = jnp.where(p0c == p1c, 1.0, 0.0)
        w3_ref[...] = jnp.concatenate([w, v, d, jnp.zeros_like(w)], axis=1)


def _tc_pairs(p0c, p1c, p0r, p1r, z_p, c_se):
    GI = NPAIR // _PT
    GJ = NPAIR // _PT
    return pl.pallas_call(
        _pairs_body,
        grid=(GI, GJ),
        in_specs=[
            pl.BlockSpec((_PT, 1), lambda i, j: (i, 0)),
            pl.BlockSpec((_PT, 1), lambda i, j: (i, 0)),
            pl.BlockSpec((_PT, 1), lambda i, j: (i, 0)),
            pl.BlockSpec((1, _PT), lambda i, j: (0, j)),
            pl.BlockSpec((1, _PT), lambda i, j: (0, j)),
            pl.BlockSpec((1, _PT), lambda i, j: (0, jnp.minimum(j, B // _PT - 1))),
            pl.BlockSpec((1, _CT), lambda i, j: (0, j)),
        ],
        out_specs=[
            pl.BlockSpec((_PT, 1), lambda i, j: (i, 0)),
            pl.BlockSpec((_PT, 1), lambda i, j: (i, 0)),
            pl.BlockSpec((_CT, 1), lambda i, j: (i, 0)),
            pl.BlockSpec((_PT, 4), lambda i, j: (i, 0)),
        ],
        out_shape=[
            jax.ShapeDtypeStruct((NPAIR, 1), f32),
            jax.ShapeDtypeStruct((NPAIR, 1), f32),
            jax.ShapeDtypeStruct((B, 1), f32),
            jax.ShapeDtypeStruct((NPAIR, 4), f32),
        ],
        compiler_params=pltpu.CompilerParams(
            dimension_semantics=("arbitrary", "arbitrary")),
    )(p0c, p1c, z_p, p0r, p1r, c_se, c_se)


_AT = 1024


def _agg_body(p0r_ref, w3_ref, agg_ref):
    t = pl.program_id(0)
    k = pl.program_id(1)
    bf16 = jnp.bfloat16
    rows = lax.broadcasted_iota(i32, (_AT, 1), 0) + t * _AT
    oht = jnp.where(rows == p0r_ref[...], 1.0, 0.0).astype(bf16)
    part = lax.dot_general(oht, w3_ref[...].astype(bf16),
                           (((1,), (0,)), ((), ())),
                           preferred_element_type=f32)

    @pl.when(k == 0)
    def _():
        agg_ref[...] = part

    @pl.when(k != 0)
    def _():
        agg_ref[...] += part


def _tc_agg(p0r, w3):
    GT = B // _AT
    GK = NPAIR // _PT
    return pl.pallas_call(
        _agg_body,
        grid=(GT, GK),
        in_specs=[
            pl.BlockSpec((1, _PT), lambda t, k: (0, k)),
            pl.BlockSpec((_PT, 4), lambda t, k: (k, 0)),
        ],
        out_specs=pl.BlockSpec((_AT, 4), lambda t, k: (t, 0)),
        out_shape=jax.ShapeDtypeStruct((B, 4), f32),
        compiler_params=pltpu.CompilerParams(
            dimension_semantics=("arbitrary", "arbitrary")),
    )(p0r, w3)


def _final_body(a_ref, bv_ref, m_ref, rse_ref, rsz_ref, zd_ref, lsec_ref,
                out_ref):
    a = a_ref[...]
    bv = bv_ref[...]
    m = m_ref[...]
    lse_row = jnp.log(rse_ref[...])
    rs_z = rsz_ref[...]
    z_diag = zd_ref[...]
    lse_col = lsec_ref[...]
    s_col = jnp.sum(lse_col)

    w_d = 1.0 / (1.0 + m)
    num_pos = a + w_d
    loss_pos = bv - a * lse_row + w_d * (2.0 * z_diag - lse_row - lse_col)
    rowsum_ls = 2.0 * rs_z - float(B) * lse_row - s_col
    loss_neg = rowsum_ls - loss_pos
    num_neg = float(B) - num_pos
    loss = -jnp.sum(loss_pos / num_pos + loss_neg / num_neg) / float(B)
    out_ref[...] = loss * jnp.ones((1, 1), f32)


def _tc_final(agg, r_se, rs_z, z_diag, lse_col_c):
    lane = (32, 128)
    args = [
        agg[:, 0:1].reshape(lane), agg[:, 1:2].reshape(lane),
        agg[:, 2:3].reshape(lane), r_se.reshape(lane), rs_z.reshape(lane),
        z_diag.reshape(lane), lse_col_c.reshape(lane),
    ]
    return pl.pallas_call(
        _final_body,
        out_shape=jax.ShapeDtypeStruct((1, 1), f32),
    )(*args)


def kernel(user_ids, item_ids, exp_ids, pos_indices, user_table, item_table,
           exp_table):
    uid = user_ids.astype(i32)
    iid = item_ids.astype(i32)
    eid = exp_ids.astype(i32)
    p0 = pos_indices[:, 0].astype(i32)
    p1 = pos_indices[:, 1].astype(i32)

    pu, pi_, pe = _sc_stripe(uid, iid, eid, user_table.T, item_table.T,
                             exp_table.T)
    u2, i2, e2 = _sc_pair(p0, p1, pu, pi_, pe)

    p0c = p0.reshape(NPAIR, 1)
    p1c = p1.reshape(NPAIR, 1)
    ui_n, e_n, z_diag, z_p = _tc_prep(
        pu.reshape(B, D), pi_.reshape(B, D), pe.reshape(B, D),
        u2, i2, e2, p0c, p1c)
    r_se, rs_z, c_se = _tc_zpass(ui_n, e_n)

    p0c = p0.reshape(NPAIR, 1)
    p1c = p1.reshape(NPAIR, 1)
    p0r = p0.reshape(1, NPAIR)
    p1r = p1.reshape(1, NPAIR)
    cnt, lse_p, lse_col_c, w3 = _tc_pairs(p0c, p1c, p0r, p1r, z_p, c_se)
    agg = _tc_agg(p0r, w3)
    out = _tc_final(agg, r_se, rs_z, z_diag, lse_col_c)
    return out[0, 0]


# speedup vs baseline: 3.3426x; 1.5586x over previous
import jax
import jax.numpy as jnp
from jax import lax
from jax.experimental import pallas as pl
from jax.experimental.pallas import tpu as pltpu
from jax.experimental.pallas import tpu_sc as plsc

MU = 0.5
TEMP = 0.1
B = 4096
D = 64
NPAIR = 8192
EPS = 1e-12

NC = 2
NS = 16
NL = 16
NW = NC * NS
CHUNK = 128

f32 = jnp.float32
i32 = jnp.int32


D2 = 2 * D


def _sc_mesh():
    return plsc.VectorSubcoreMesh(core_axis_name="c", subcore_axis_name="s",
                                  num_cores=NC, num_subcores=NS)


NBUF = 4


def _sc_stripe_body(uid_h, iid_h, eid_h, tu_h, ti_h, te_h,
                    pu_h, pi_h, pe_h, idx_v, t0_v, t1_v, t2_v, t3_v,
                    pack_v, s0, s1, s2, s3):
    wid = lax.axis_index("s") * NC + lax.axis_index("c")
    base = wid * (B // NW)
    lanes = lax.broadcasted_iota(i32, (NL,), 0)
    bufs = (t0_v, t1_v, t2_v, t3_v)
    sems = (s0, s1, s2, s3)
    for ids_h, t_h, o_h in ((uid_h, tu_h, pu_h), (iid_h, ti_h, pi_h),
                            (eid_h, te_h, pe_h)):
        pltpu.sync_copy(ids_h.at[pl.ds(base, CHUNK)], idx_v.at[pl.ds(0, CHUNK)])

        def issue(k, buf, sem):
            i = idx_v[pl.ds(k, NL)][0]
            col0 = pl.multiple_of((i >> 7) << 7, D2)
            pltpu.async_copy(t_h.at[:, pl.ds(col0, D2)], buf, sem)

        for b in range(NBUF):
            issue(b, bufs[b], sems[b])

        def grp(g, _):
            for b in range(NBUF):
                k = g * NBUF + b
                pltpu.make_async_copy(t_h.at[:, pl.ds(0, D2)], bufs[b],
                                      sems[b]).wait()
                i = idx_v[pl.ds(k, NL)][0]
                col0 = pl.multiple_of((i >> 7) << 7, D2)
                ii = jnp.broadcast_to(i - col0, (NL,))
                p = k >> 1
                off = (k & 1) * D
                for g2 in range(D // NL):
                    vals = plsc.load_gather(bufs[b], [lanes + g2 * NL, ii])
                    pack_v[p, pl.ds(off + g2 * NL, NL)] = vals

                @pl.when(k + NBUF < CHUNK)
                def _():
                    issue(k + NBUF, bufs[b], sems[b])
            return 0

        lax.fori_loop(0, CHUNK // NBUF, grp, 0)
        pltpu.sync_copy(pack_v, o_h.at[pl.ds(wid * (CHUNK // 2), CHUNK // 2)])


def _sc_stripe(uid, iid, eid, tu, ti, te):
    out_type = (
        jax.ShapeDtypeStruct((B // 2, D2), f32),
        jax.ShapeDtypeStruct((B // 2, D2), f32),
        jax.ShapeDtypeStruct((B // 2, D2), f32),
    )
    scratch_types = [
        pltpu.VMEM((CHUNK + NL,), i32),
        pltpu.VMEM((D, D2), f32),
        pltpu.VMEM((D, D2), f32),
        pltpu.VMEM((D, D2), f32),
        pltpu.VMEM((D, D2), f32),
        pltpu.VMEM((CHUNK // 2, D2), f32),
        pltpu.SemaphoreType.DMA,
        pltpu.SemaphoreType.DMA,
        pltpu.SemaphoreType.DMA,
        pltpu.SemaphoreType.DMA,
    ]
    fn = pl.kernel(_sc_stripe_body, out_type=out_type, mesh=_sc_mesh(),
                   scratch_types=scratch_types,
                   compiler_params=pltpu.CompilerParams(
                       needs_layout_passes=False,
                       use_tc_tiling_on_sc=True))
    return fn(uid, iid, eid, tu, ti, te)


def _sc_pair_body(p0_h, p1_h, pu_h, pi_h, pe_h, u2_h, i2_h, e2_h,
                  idx_v, idxs_v, rows_v, sem):
    wid = lax.axis_index("s") * NC + lax.axis_index("c")
    for p_h, src_h, out_h in ((p0_h, pu_h, u2_h), (p0_h, pi_h, i2_h),
                              (p1_h, pe_h, e2_h)):
        for c in range(NPAIR // NW // CHUNK):
            base = wid * (NPAIR // NW) + c * CHUNK
            pltpu.sync_copy(p_h.at[pl.ds(base, CHUNK)], idx_v)
            for g in range(CHUNK // NL):
                idxs_v[pl.ds(g * NL, NL)] = idx_v[pl.ds(g * NL, NL)] >> 1
            pltpu.async_copy(src_h.at[idxs_v], rows_v, sem).wait()
            pltpu.sync_copy(rows_v, out_h.at[pl.ds(base, CHUNK)])


def _sc_pair(p0, p1, pu, pi_, pe):
    out_type = (
        jax.ShapeDtypeStruct((NPAIR, D2), f32),
        jax.ShapeDtypeStruct((NPAIR, D2), f32),
        jax.ShapeDtypeStruct((NPAIR, D2), f32),
    )
    scratch_types = [
        pltpu.VMEM((CHUNK,), i32),
        pltpu.VMEM((CHUNK,), i32),
        pltpu.VMEM((CHUNK, D2), f32),
        pltpu.SemaphoreType.DMA,
    ]
    fn = pl.kernel(_sc_pair_body, out_type=out_type, mesh=_sc_mesh(),
                   scratch_types=scratch_types,
                   compiler_params=pltpu.CompilerParams(
                       needs_layout_passes=False,
                       use_tc_tiling_on_sc=True))
    return fn(p0, p1, pu, pi_, pe)


def _halfsel(packed, par):
    return jnp.where((par & 1) == 1, packed[:, D:], packed[:, :D])


def _prep_body(u_ref, i_ref, e_ref, u2_ref, i2_ref, e2_ref,
               p0c_ref, p1c_ref, ui_ref, en_ref, zd_ref, zp_ref):
    mix = MU * u_ref[...] + (1.0 - MU) * i_ref[...]
    nm = jnp.sqrt(jnp.sum(mix * mix, axis=1, keepdims=True))
    ui = mix / jnp.maximum(nm, EPS)
    ex = e_ref[...]
    ne = jnp.sqrt(jnp.sum(ex * ex, axis=1, keepdims=True))
    en = ex / jnp.maximum(ne, EPS)
    ui_ref[...] = ui
    en_ref[...] = en
    zd_ref[...] = jnp.sum(ui * en, axis=1, keepdims=True) / TEMP

    u2 = _halfsel(u2_ref[...], p0c_ref[...])
    i2 = _halfsel(i2_ref[...], p0c_ref[...])
    e2 = _halfsel(e2_ref[...], p1c_ref[...])
    mix2 = MU * u2 + (1.0 - MU) * i2
    n1 = jnp.maximum(jnp.sqrt(jnp.sum(mix2 * mix2, axis=1, keepdims=True)), EPS)
    n2 = jnp.maximum(jnp.sqrt(jnp.sum(e2 * e2, axis=1, keepdims=True)), EPS)
    dt = jnp.sum(mix2 * e2, axis=1, keepdims=True)
    zp_ref[...] = dt / (n1 * n2) / TEMP


def _tc_prep(u_rows, i_rows, e_rows, u2, i2, e2, p0c, p1c):
    G = 8
    RB = B // G
    PB = NPAIR // G
    return pl.pallas_call(
        _prep_body,
        grid=(G,),
        in_specs=[
            pl.BlockSpec((RB, D), lambda g: (g, 0)),
            pl.BlockSpec((RB, D), lambda g: (g, 0)),
            pl.BlockSpec((RB, D), lambda g: (g, 0)),
            pl.BlockSpec((PB, D2), lambda g: (g, 0)),
            pl.BlockSpec((PB, D2), lambda g: (g, 0)),
            pl.BlockSpec((PB, D2), lambda g: (g, 0)),
            pl.BlockSpec((PB, 1), lambda g: (g, 0)),
            pl.BlockSpec((PB, 1), lambda g: (g, 0)),
        ],
        out_specs=[
            pl.BlockSpec((RB, D), lambda g: (g, 0)),
            pl.BlockSpec((RB, D), lambda g: (g, 0)),
            pl.BlockSpec((RB, 1), lambda g: (g, 0)),
            pl.BlockSpec((PB, 1), lambda g: (g, 0)),
        ],
        out_shape=[
            jax.ShapeDtypeStruct((B, D), f32),
            jax.ShapeDtypeStruct((B, D), f32),
            jax.ShapeDtypeStruct((B, 1), f32),
            jax.ShapeDtypeStruct((NPAIR, 1), f32),
        ],
    )(u_rows, i_rows, e_rows, u2, i2, e2, p0c, p1c)


_ZT = 512


def _zpass_body(ui_ref, en_ref, rse_ref, rsz_ref, cse_ref):
    i = pl.program_id(0)
    j = pl.program_id(1)
    zt = lax.dot_general(ui_ref[...].astype(jnp.bfloat16),
                         en_ref[...].astype(jnp.bfloat16),
                         (((1,), (1,)), ((), ())),
                         preferred_element_type=f32) * (1.0 / TEMP)
    ez = jnp.exp(zt)
    rse_part = jnp.sum(ez, axis=1, keepdims=True)
    rsz_part = jnp.sum(zt, axis=1, keepdims=True)
    cse_part = jnp.sum(ez, axis=0, keepdims=True)

    @pl.when(j == 0)
    def _():
        rse_ref[...] = rse_part
        rsz_ref[...] = rsz_part

    @pl.when(j != 0)
    def _():
        rse_ref[...] += rse_part
        rsz_ref[...] += rsz_part

    @pl.when(i == 0)
    def _():
        cse_ref[:, pl.ds(j * _ZT, _ZT)] = cse_part

    @pl.when(i != 0)
    def _():
        cse_ref[:, pl.ds(j * _ZT, _ZT)] += cse_part


def _tc_zpass(ui_n, e_n):
    G = B // _ZT
    return pl.pallas_call(
        _zpass_body,
        grid=(G, G),
        in_specs=[
            pl.BlockSpec((_ZT, D), lambda i, j: (i, 0)),
            pl.BlockSpec((_ZT, D), lambda i, j: (j, 0)),
        ],
        out_specs=[
            pl.BlockSpec((_ZT, 1), lambda i, j: (i, 0)),
            pl.BlockSpec((_ZT, 1), lambda i, j: (i, 0)),
            pl.BlockSpec((1, B), lambda i, j: (0, 0)),
        ],
        out_shape=[
            jax.ShapeDtypeStruct((B, 1), f32),
            jax.ShapeDtypeStruct((B, 1), f32),
            jax.ShapeDtypeStruct((1, B), f32),
        ],
        compiler_params=pltpu.CompilerParams(
            dimension_semantics=("arbitrary", "arbitrary")),
    )(ui_n, e_n)


_PT = 1024
_CT = 512


def _pairs_body(p0c_ref, p1c_ref, zp_ref, p0r_ref, p1r_ref, cseg_ref,
                cset_ref, cnt_ref, lsep_ref, lsec_ref, w3_ref):
    j = pl.program_id(1)
    nj = pl.num_programs(1)

    p0c = p0c_ref[...]
    p1c = p1c_ref[...]
    keyc = p0c * B + p1c
    keyr = p0r_ref[...] * B + p1r_ref[...]
    eq = jnp.where(keyc == keyr, 1.0, 0.0)
    cnt_part = jnp.sum(eq, axis=1, keepdims=True)

    @pl.when(j == 0)
    def _():
        cnt_ref[...] = cnt_part + jnp.where(p0c == p1c, 1.0, 0.0)

    @pl.when(j != 0)
    def _():
        cnt_ref[...] += cnt_part

    def g_part():
        lse_g = jnp.log(cseg_ref[...])
        cols_g = lax.broadcasted_iota(i32, (1, _PT), 1) + j * _PT
        return jnp.sum(jnp.where(p1c == cols_g, lse_g, 0.0),
                       axis=1, keepdims=True)

    @pl.when(j == 0)
    def _():
        lsep_ref[...] = g_part()

    @pl.when(jnp.logical_and(j != 0, j < B // _PT))
    def _():
        lsep_ref[...] += g_part()

    i = pl.program_id(0)
    lse_t = jnp.log(cset_ref[...])
    rows_t = lax.broadcasted_iota(i32, (_CT, 1), 0) + i * _CT
    cols_t = lax.broadcasted_iota(i32, (1, _CT), 1) + j * _CT
    t_part = jnp.sum(jnp.where(rows_t == cols_t, lse_t, 0.0),
                     axis=1, keepdims=True)

    @pl.when(j == 0)
    def _():
        lsec_ref[...] = t_part

    @pl.when(j != 0)
    def _():
        lsec_ref[...] += t_part

    @pl.when(j == nj - 1)
    def _():
        w = 1.0 / cnt_ref[...]
        v = w * (2.0 * zp_ref[...] - lsep_ref[...])
        d = jnp.where(p0c == p1c, 1.0, 0.0)
        w3_ref[...] = jnp.concatenate([w, v, d, jnp.zeros_like(w)], axis=1)


def _tc_pairs(p0c, p1c, p0r, p1r, z_p, c_se):
    GI = NPAIR // _PT
    GJ = NPAIR // _PT
    return pl.pallas_call(
        _pairs_body,
        grid=(GI, GJ),
        in_specs=[
            pl.BlockSpec((_PT, 1), lambda i, j: (i, 0)),
            pl.BlockSpec((_PT, 1), lambda i, j: (i, 0)),
            pl.BlockSpec((_PT, 1), lambda i, j: (i, 0)),
            pl.BlockSpec((1, _PT), lambda i, j: (0, j)),
            pl.BlockSpec((1, _PT), lambda i, j: (0, j)),
            pl.BlockSpec((1, _PT), lambda i, j: (0, jnp.minimum(j, B // _PT - 1))),
            pl.BlockSpec((1, _CT), lambda i, j: (0, j)),
        ],
        out_specs=[
            pl.BlockSpec((_PT, 1), lambda i, j: (i, 0)),
            pl.BlockSpec((_PT, 1), lambda i, j: (i, 0)),
            pl.BlockSpec((_CT, 1), lambda i, j: (i, 0)),
            pl.BlockSpec((_PT, 4), lambda i, j: (i, 0)),
        ],
        out_shape=[
            jax.ShapeDtypeStruct((NPAIR, 1), f32),
            jax.ShapeDtypeStruct((NPAIR, 1), f32),
            jax.ShapeDtypeStruct((B, 1), f32),
            jax.ShapeDtypeStruct((NPAIR, 4), f32),
        ],
        compiler_params=pltpu.CompilerParams(
            dimension_semantics=("arbitrary", "arbitrary")),
    )(p0c, p1c, z_p, p0r, p1r, c_se, c_se)


_AT = 1024


def _agg_body(p0r_ref, w3_ref, agg_ref):
    t = pl.program_id(0)
    k = pl.program_id(1)
    bf16 = jnp.bfloat16
    rows = lax.broadcasted_iota(i32, (_AT, 1), 0) + t * _AT
    oht = jnp.where(rows == p0r_ref[...], 1.0, 0.0).astype(bf16)
    part = lax.dot_general(oht, w3_ref[...].astype(bf16),
                           (((1,), (0,)), ((), ())),
                           preferred_element_type=f32)

    @pl.when(k == 0)
    def _():
        agg_ref[...] = part

    @pl.when(k != 0)
    def _():
        agg_ref[...] += part


def _tc_agg(p0r, w3):
    GT = B // _AT
    GK = NPAIR // _PT
    return pl.pallas_call(
        _agg_body,
        grid=(GT, GK),
        in_specs=[
            pl.BlockSpec((1, _PT), lambda t, k: (0, k)),
            pl.BlockSpec((_PT, 4), lambda t, k: (k, 0)),
        ],
        out_specs=pl.BlockSpec((_AT, 4), lambda t, k: (t, 0)),
        out_shape=jax.ShapeDtypeStruct((B, 4), f32),
        compiler_params=pltpu.CompilerParams(
            dimension_semantics=("arbitrary", "arbitrary")),
    )(p0r, w3)


def _final_body(a_ref, bv_ref, m_ref, rse_ref, rsz_ref, zd_ref, lsec_ref,
                out_ref):
    a = a_ref[...]
    bv = bv_ref[...]
    m = m_ref[...]
    lse_row = jnp.log(rse_ref[...])
    rs_z = rsz_ref[...]
    z_diag = zd_ref[...]
    lse_col = lsec_ref[...]
    s_col = jnp.sum(lse_col)

    w_d = 1.0 / (1.0 + m)
    num_pos = a + w_d
    loss_pos = bv - a * lse_row + w_d * (2.0 * z_diag - lse_row - lse_col)
    rowsum_ls = 2.0 * rs_z - float(B) * lse_row - s_col
    loss_neg = rowsum_ls - loss_pos
    num_neg = float(B) - num_pos
    loss = -jnp.sum(loss_pos / num_pos + loss_neg / num_neg) / float(B)
    out_ref[...] = loss * jnp.ones((1, 1), f32)


def _tc_final(agg, r_se, rs_z, z_diag, lse_col_c):
    lane = (32, 128)
    args = [
        agg[:, 0:1].reshape(lane), agg[:, 1:2].reshape(lane),
        agg[:, 2:3].reshape(lane), r_se.reshape(lane), rs_z.reshape(lane),
        z_diag.reshape(lane), lse_col_c.reshape(lane),
    ]
    return pl.pallas_call(
        _final_body,
        out_shape=jax.ShapeDtypeStruct((1, 1), f32),
    )(*args)


def kernel(user_ids, item_ids, exp_ids, pos_indices, user_table, item_table,
           exp_table):
    uid = user_ids.astype(i32)
    iid = item_ids.astype(i32)
    eid = exp_ids.astype(i32)
    p0 = pos_indices[:, 0].astype(i32)
    p1 = pos_indices[:, 1].astype(i32)

    pu, pi_, pe = _sc_stripe(uid, iid, eid, user_table.T, item_table.T,
                             exp_table.T)
    u2, i2, e2 = _sc_pair(p0, p1, pu, pi_, pe)

    p0c = p0.reshape(NPAIR, 1)
    p1c = p1.reshape(NPAIR, 1)
    ui_n, e_n, z_diag, z_p = _tc_prep(
        pu.reshape(B, D), pi_.reshape(B, D), pe.reshape(B, D),
        u2, i2, e2, p0c, p1c)
    r_se, rs_z, c_se = _tc_zpass(ui_n, e_n)

    p0c = p0.reshape(NPAIR, 1)
    p1c = p1.reshape(NPAIR, 1)
    p0r = p0.reshape(1, NPAIR)
    p1r = p1.reshape(1, NPAIR)
    cnt, lse_p, lse_col_c, w3 = _tc_pairs(p0c, p1c, p0r, p1r, z_p, c_se)
    agg = _tc_agg(p0r, w3)
    out = _tc_final(agg, r_se, rs_z, z_diag, lse_col_c)
    return out[0, 0]


# speedup vs baseline: 3.3928x; 1.0150x over previous
import jax
import jax.numpy as jnp
from jax import lax
from jax.experimental import pallas as pl
from jax.experimental.pallas import tpu as pltpu
from jax.experimental.pallas import tpu_sc as plsc

MU = 0.5
TEMP = 0.1
B = 4096
D = 64
NPAIR = 8192
EPS = 1e-12

NC = 2
NS = 16
NL = 16
NW = NC * NS
CHUNK = 128

f32 = jnp.float32
i32 = jnp.int32


D2 = 2 * D


def _sc_mesh():
    return plsc.VectorSubcoreMesh(core_axis_name="c", subcore_axis_name="s",
                                  num_cores=NC, num_subcores=NS)


NBUF = 4


def _sc_stripe_body(uid_h, iid_h, eid_h, tu_h, ti_h, te_h,
                    pu_h, pi_h, pe_h, idx_v, t0_v, t1_v, t2_v, t3_v,
                    pack_v, s0, s1, s2, s3):
    wid = lax.axis_index("s") * NC + lax.axis_index("c")
    base = wid * (B // NW)
    lanes = lax.broadcasted_iota(i32, (NL,), 0)
    bufs = (t0_v, t1_v, t2_v, t3_v)
    sems = (s0, s1, s2, s3)
    for ids_h, t_h, o_h in ((uid_h, tu_h, pu_h), (iid_h, ti_h, pi_h),
                            (eid_h, te_h, pe_h)):
        pltpu.sync_copy(ids_h.at[pl.ds(base, CHUNK)], idx_v.at[pl.ds(0, CHUNK)])

        def issue(k, buf, sem):
            i = idx_v[pl.ds(k, NL)][0]
            col0 = pl.multiple_of((i >> 7) << 7, D2)
            pltpu.async_copy(t_h.at[:, pl.ds(col0, D2)], buf, sem)

        for b in range(NBUF):
            issue(b, bufs[b], sems[b])

        def grp(g, _):
            for b in range(NBUF):
                k = g * NBUF + b
                pltpu.make_async_copy(t_h.at[:, pl.ds(0, D2)], bufs[b],
                                      sems[b]).wait()
                i = idx_v[pl.ds(k, NL)][0]
                col0 = pl.multiple_of((i >> 7) << 7, D2)
                ii = jnp.broadcast_to(i - col0, (NL,))
                p = k >> 1
                off = (k & 1) * D
                for g2 in range(D // NL):
                    vals = plsc.load_gather(bufs[b], [lanes + g2 * NL, ii])
                    pack_v[p, pl.ds(off + g2 * NL, NL)] = vals

                @pl.when(k + NBUF < CHUNK)
                def _():
                    issue(k + NBUF, bufs[b], sems[b])
            return 0

        lax.fori_loop(0, CHUNK // NBUF, grp, 0)
        pltpu.sync_copy(pack_v, o_h.at[pl.ds(wid * (CHUNK // 2), CHUNK // 2)])


def _sc_stripe(uid, iid, eid, tu, ti, te):
    out_type = (
        jax.ShapeDtypeStruct((B // 2, D2), f32),
        jax.ShapeDtypeStruct((B // 2, D2), f32),
        jax.ShapeDtypeStruct((B // 2, D2), f32),
    )
    scratch_types = [
        pltpu.VMEM((CHUNK + NL,), i32),
        pltpu.VMEM((D, D2), f32),
        pltpu.VMEM((D, D2), f32),
        pltpu.VMEM((D, D2), f32),
        pltpu.VMEM((D, D2), f32),
        pltpu.VMEM((CHUNK // 2, D2), f32),
        pltpu.SemaphoreType.DMA,
        pltpu.SemaphoreType.DMA,
        pltpu.SemaphoreType.DMA,
        pltpu.SemaphoreType.DMA,
    ]
    fn = pl.kernel(_sc_stripe_body, out_type=out_type, mesh=_sc_mesh(),
                   scratch_types=scratch_types,
                   compiler_params=pltpu.CompilerParams(
                       needs_layout_passes=False,
                       use_tc_tiling_on_sc=True))
    return fn(uid, iid, eid, tu, ti, te)


def _sc_pair_body(p0_h, p1_h, pu_h, pi_h, pe_h, u2_h, i2_h, e2_h,
                  idx_v, idxs_v, rows_v, sem):
    wid = lax.axis_index("s") * NC + lax.axis_index("c")
    for p_h, src_h, out_h in ((p0_h, pu_h, u2_h), (p0_h, pi_h, i2_h),
                              (p1_h, pe_h, e2_h)):
        for c in range(NPAIR // NW // CHUNK):
            base = wid * (NPAIR // NW) + c * CHUNK
            pltpu.sync_copy(p_h.at[pl.ds(base, CHUNK)], idx_v)
            for g in range(CHUNK // NL):
                idxs_v[pl.ds(g * NL, NL)] = idx_v[pl.ds(g * NL, NL)] >> 1
            pltpu.async_copy(src_h.at[idxs_v], rows_v, sem).wait()
            pltpu.sync_copy(rows_v, out_h.at[pl.ds(base, CHUNK)])


def _sc_pair(p0, p1, pu, pi_, pe):
    out_type = (
        jax.ShapeDtypeStruct((NPAIR, D2), f32),
        jax.ShapeDtypeStruct((NPAIR, D2), f32),
        jax.ShapeDtypeStruct((NPAIR, D2), f32),
    )
    scratch_types = [
        pltpu.VMEM((CHUNK,), i32),
        pltpu.VMEM((CHUNK,), i32),
        pltpu.VMEM((CHUNK, D2), f32),
        pltpu.SemaphoreType.DMA,
    ]
    fn = pl.kernel(_sc_pair_body, out_type=out_type, mesh=_sc_mesh(),
                   scratch_types=scratch_types,
                   compiler_params=pltpu.CompilerParams(
                       needs_layout_passes=False,
                       use_tc_tiling_on_sc=True))
    return fn(p0, p1, pu, pi_, pe)


def _halfsel(packed, par):
    return jnp.where((par & 1) == 1, packed[:, D:], packed[:, :D])


def _prep_body(u_ref, i_ref, e_ref, u2_ref, i2_ref, e2_ref,
               p0c_ref, p1c_ref, ui_ref, en_ref, zd_ref, zp_ref):
    mix = MU * u_ref[...] + (1.0 - MU) * i_ref[...]
    nm = jnp.sqrt(jnp.sum(mix * mix, axis=1, keepdims=True))
    ui = mix / jnp.maximum(nm, EPS)
    ex = e_ref[...]
    ne = jnp.sqrt(jnp.sum(ex * ex, axis=1, keepdims=True))
    en = ex / jnp.maximum(ne, EPS)
    ui_ref[...] = ui
    en_ref[...] = en
    zd_ref[...] = jnp.sum(ui * en, axis=1, keepdims=True) / TEMP

    u2 = _halfsel(u2_ref[...], p0c_ref[...])
    i2 = _halfsel(i2_ref[...], p0c_ref[...])
    e2 = _halfsel(e2_ref[...], p1c_ref[...])
    mix2 = MU * u2 + (1.0 - MU) * i2
    n1 = jnp.maximum(jnp.sqrt(jnp.sum(mix2 * mix2, axis=1, keepdims=True)), EPS)
    n2 = jnp.maximum(jnp.sqrt(jnp.sum(e2 * e2, axis=1, keepdims=True)), EPS)
    dt = jnp.sum(mix2 * e2, axis=1, keepdims=True)
    zp_ref[...] = dt / (n1 * n2) / TEMP


def _tc_prep(u_rows, i_rows, e_rows, u2, i2, e2, p0c, p1c):
    G = 8
    RB = B // G
    PB = NPAIR // G
    return pl.pallas_call(
        _prep_body,
        grid=(G,),
        in_specs=[
            pl.BlockSpec((RB, D), lambda g: (g, 0)),
            pl.BlockSpec((RB, D), lambda g: (g, 0)),
            pl.BlockSpec((RB, D), lambda g: (g, 0)),
            pl.BlockSpec((PB, D2), lambda g: (g, 0)),
            pl.BlockSpec((PB, D2), lambda g: (g, 0)),
            pl.BlockSpec((PB, D2), lambda g: (g, 0)),
            pl.BlockSpec((PB, 1), lambda g: (g, 0)),
            pl.BlockSpec((PB, 1), lambda g: (g, 0)),
        ],
        out_specs=[
            pl.BlockSpec((RB, D), lambda g: (g, 0)),
            pl.BlockSpec((RB, D), lambda g: (g, 0)),
            pl.BlockSpec((RB, 1), lambda g: (g, 0)),
            pl.BlockSpec((PB, 1), lambda g: (g, 0)),
        ],
        out_shape=[
            jax.ShapeDtypeStruct((B, D), f32),
            jax.ShapeDtypeStruct((B, D), f32),
            jax.ShapeDtypeStruct((B, 1), f32),
            jax.ShapeDtypeStruct((NPAIR, 1), f32),
        ],
    )(u_rows, i_rows, e_rows, u2, i2, e2, p0c, p1c)


_ZT = 512


def _zpass_body(ui_ref, en_ref, rse_ref, rsz_ref, cse_ref):
    i = pl.program_id(0)
    j = pl.program_id(1)
    zt = lax.dot_general(ui_ref[...].astype(jnp.bfloat16),
                         en_ref[...].astype(jnp.bfloat16),
                         (((1,), (1,)), ((), ())),
                         preferred_element_type=f32) * (1.0 / TEMP)
    ez = jnp.exp(zt)
    rse_part = jnp.sum(ez, axis=1, keepdims=True)
    rsz_part = jnp.sum(zt, axis=1, keepdims=True)
    cse_part = jnp.sum(ez, axis=0, keepdims=True)

    @pl.when(j == 0)
    def _():
        rse_ref[...] = rse_part
        rsz_ref[...] = rsz_part

    @pl.when(j != 0)
    def _():
        rse_ref[...] += rse_part
        rsz_ref[...] += rsz_part

    @pl.when(i == 0)
    def _():
        cse_ref[:, pl.ds(j * _ZT, _ZT)] = cse_part

    @pl.when(i != 0)
    def _():
        cse_ref[:, pl.ds(j * _ZT, _ZT)] += cse_part


def _tc_zpass(ui_n, e_n):
    G = B // _ZT
    return pl.pallas_call(
        _zpass_body,
        grid=(G, G),
        in_specs=[
            pl.BlockSpec((_ZT, D), lambda i, j: (i, 0)),
            pl.BlockSpec((_ZT, D), lambda i, j: (j, 0)),
        ],
        out_specs=[
            pl.BlockSpec((_ZT, 1), lambda i, j: (i, 0)),
            pl.BlockSpec((_ZT, 1), lambda i, j: (i, 0)),
            pl.BlockSpec((1, B), lambda i, j: (0, 0)),
        ],
        out_shape=[
            jax.ShapeDtypeStruct((B, 1), f32),
            jax.ShapeDtypeStruct((B, 1), f32),
            jax.ShapeDtypeStruct((1, B), f32),
        ],
        compiler_params=pltpu.CompilerParams(
            dimension_semantics=("arbitrary", "arbitrary")),
    )(ui_n, e_n)


_PT = 1024
_CT = 512


def _cnt_body(p0c_ref, p1c_ref, p0r_ref, p1r_ref, cnt_ref):
    j = pl.program_id(1)
    p0c = p0c_ref[...]
    p1c = p1c_ref[...]
    keyc = p0c * B + p1c
    keyr = p0r_ref[...] * B + p1r_ref[...]
    eq = jnp.where(keyc == keyr, 1.0, 0.0)
    cnt_part = jnp.sum(eq, axis=1, keepdims=True)

    @pl.when(j == 0)
    def _():
        cnt_ref[...] = cnt_part + jnp.where(p0c == p1c, 1.0, 0.0)

    @pl.when(j != 0)
    def _():
        cnt_ref[...] += cnt_part


def _tc_cnt(p0c, p1c, p0r, p1r):
    G = NPAIR // _PT
    return pl.pallas_call(
        _cnt_body,
        grid=(G, G),
        in_specs=[
            pl.BlockSpec((_PT, 1), lambda i, j: (i, 0)),
            pl.BlockSpec((_PT, 1), lambda i, j: (i, 0)),
            pl.BlockSpec((1, _PT), lambda i, j: (0, j)),
            pl.BlockSpec((1, _PT), lambda i, j: (0, j)),
        ],
        out_specs=pl.BlockSpec((_PT, 1), lambda i, j: (i, 0)),
        out_shape=jax.ShapeDtypeStruct((NPAIR, 1), f32),
        compiler_params=pltpu.CompilerParams(
            dimension_semantics=("arbitrary", "arbitrary")),
    )(p0c, p1c, p0r, p1r)


def _pairs_body(p0c_ref, p1c_ref, zp_ref, cnt_ref, cse_ref,
                lsep_ref, lsec_ref, w3_ref):
    i = pl.program_id(0)
    j = pl.program_id(1)
    nj = pl.num_programs(1)
    p0c = p0c_ref[...]
    p1c = p1c_ref[...]
    lse_g = jnp.log(cse_ref[...])

    cols_g = lax.broadcasted_iota(i32, (1, _PT), 1) + j * _PT
    g_part = jnp.sum(jnp.where(p1c == cols_g, lse_g, 0.0),
                     axis=1, keepdims=True)

    @pl.when(j == 0)
    def _():
        lsep_ref[...] = g_part

    @pl.when(j != 0)
    def _():
        lsep_ref[...] += g_part

    rows_t = lax.broadcasted_iota(i32, (_CT, 1), 0) + i * _CT
    t_part = jnp.sum(jnp.where(rows_t == cols_g, lse_g, 0.0),
                     axis=1, keepdims=True)

    @pl.when(j == 0)
    def _():
        lsec_ref[...] = t_part

    @pl.when(j != 0)
    def _():
        lsec_ref[...] += t_part

    @pl.when(j == nj - 1)
    def _():
        w = 1.0 / cnt_ref[...]
        v = w * (2.0 * zp_ref[...] - lsep_ref[...])
        d = jnp.where(p0c == p1c, 1.0, 0.0)
        w3_ref[...] = jnp.concatenate([w, v, d, jnp.zeros_like(w)], axis=1)


def _tc_pairs(p0c, p1c, z_p, cnt, c_se):
    GI = NPAIR // _PT
    GJ = B // _PT
    return pl.pallas_call(
        _pairs_body,
        grid=(GI, GJ),
        in_specs=[
            pl.BlockSpec((_PT, 1), lambda i, j: (i, 0)),
            pl.BlockSpec((_PT, 1), lambda i, j: (i, 0)),
            pl.BlockSpec((_PT, 1), lambda i, j: (i, 0)),
            pl.BlockSpec((_PT, 1), lambda i, j: (i, 0)),
            pl.BlockSpec((1, _PT), lambda i, j: (0, j)),
        ],
        out_specs=[
            pl.BlockSpec((_PT, 1), lambda i, j: (i, 0)),
            pl.BlockSpec((_CT, 1), lambda i, j: (i, 0)),
            pl.BlockSpec((_PT, 4), lambda i, j: (i, 0)),
        ],
        out_shape=[
            jax.ShapeDtypeStruct((NPAIR, 1), f32),
            jax.ShapeDtypeStruct((B, 1), f32),
            jax.ShapeDtypeStruct((NPAIR, 4), f32),
        ],
        compiler_params=pltpu.CompilerParams(
            dimension_semantics=("arbitrary", "arbitrary")),
    )(p0c, p1c, z_p, cnt, c_se)


_AT = 1024


def _agg_body(p0r_ref, w3_ref, agg_ref):
    t = pl.program_id(0)
    k = pl.program_id(1)
    bf16 = jnp.bfloat16
    rows = lax.broadcasted_iota(i32, (_AT, 1), 0) + t * _AT
    oht = jnp.where(rows == p0r_ref[...], 1.0, 0.0).astype(bf16)
    part = lax.dot_general(oht, w3_ref[...].astype(bf16),
                           (((1,), (0,)), ((), ())),
                           preferred_element_type=f32)

    @pl.when(k == 0)
    def _():
        agg_ref[...] = part

    @pl.when(k != 0)
    def _():
        agg_ref[...] += part


def _tc_agg(p0r, w3):
    GT = B // _AT
    GK = NPAIR // _PT
    return pl.pallas_call(
        _agg_body,
        grid=(GT, GK),
        in_specs=[
            pl.BlockSpec((1, _PT), lambda t, k: (0, k)),
            pl.BlockSpec((_PT, 4), lambda t, k: (k, 0)),
        ],
        out_specs=pl.BlockSpec((_AT, 4), lambda t, k: (t, 0)),
        out_shape=jax.ShapeDtypeStruct((B, 4), f32),
        compiler_params=pltpu.CompilerParams(
            dimension_semantics=("arbitrary", "arbitrary")),
    )(p0r, w3)


def _final_body(a_ref, bv_ref, m_ref, rse_ref, rsz_ref, zd_ref, lsec_ref,
                out_ref):
    a = a_ref[...]
    bv = bv_ref[...]
    m = m_ref[...]
    lse_row = jnp.log(rse_ref[...])
    rs_z = rsz_ref[...]
    z_diag = zd_ref[...]
    lse_col = lsec_ref[...]
    s_col = jnp.sum(lse_col)

    w_d = 1.0 / (1.0 + m)
    num_pos = a + w_d
    loss_pos = bv - a * lse_row + w_d * (2.0 * z_diag - lse_row - lse_col)
    rowsum_ls = 2.0 * rs_z - float(B) * lse_row - s_col
    loss_neg = rowsum_ls - loss_pos
    num_neg = float(B) - num_pos
    loss = -jnp.sum(loss_pos / num_pos + loss_neg / num_neg) / float(B)
    out_ref[...] = loss * jnp.ones((1, 1), f32)


def _tc_final(agg, r_se, rs_z, z_diag, lse_col_c):
    lane = (32, 128)
    args = [
        agg[:, 0:1].reshape(lane), agg[:, 1:2].reshape(lane),
        agg[:, 2:3].reshape(lane), r_se.reshape(lane), rs_z.reshape(lane),
        z_diag.reshape(lane), lse_col_c.reshape(lane),
    ]
    return pl.pallas_call(
        _final_body,
        out_shape=jax.ShapeDtypeStruct((1, 1), f32),
    )(*args)


def kernel(user_ids, item_ids, exp_ids, pos_indices, user_table, item_table,
           exp_table):
    uid = user_ids.astype(i32)
    iid = item_ids.astype(i32)
    eid = exp_ids.astype(i32)
    p0 = pos_indices[:, 0].astype(i32)
    p1 = pos_indices[:, 1].astype(i32)

    pu, pi_, pe = _sc_stripe(uid, iid, eid, user_table.T, item_table.T,
                             exp_table.T)
    u2, i2, e2 = _sc_pair(p0, p1, pu, pi_, pe)

    p0c = p0.reshape(NPAIR, 1)
    p1c = p1.reshape(NPAIR, 1)
    ui_n, e_n, z_diag, z_p = _tc_prep(
        pu.reshape(B, D), pi_.reshape(B, D), pe.reshape(B, D),
        u2, i2, e2, p0c, p1c)
    r_se, rs_z, c_se = _tc_zpass(ui_n, e_n)

    p0r = p0.reshape(1, NPAIR)
    p1r = p1.reshape(1, NPAIR)
    cnt = _tc_cnt(p0c, p1c, p0r, p1r)
    lse_p, lse_col_c, w3 = _tc_pairs(p0c, p1c, z_p, cnt, c_se)
    agg = _tc_agg(p0r, w3)
    out = _tc_final(agg, r_se, rs_z, z_diag, lse_col_c)
    return out[0, 0]


# speedup vs baseline: 3.4052x; 1.0037x over previous
import jax
import jax.numpy as jnp
from jax import lax
from jax.experimental import pallas as pl
from jax.experimental.pallas import tpu as pltpu
from jax.experimental.pallas import tpu_sc as plsc

MU = 0.5
TEMP = 0.1
B = 4096
D = 64
NPAIR = 8192
EPS = 1e-12

NC = 2
NS = 16
NL = 16
NW = NC * NS
CHUNK = 128

f32 = jnp.float32
i32 = jnp.int32


D2 = 2 * D


def _sc_mesh():
    return plsc.VectorSubcoreMesh(core_axis_name="c", subcore_axis_name="s",
                                  num_cores=NC, num_subcores=NS)


NBUF = 4


def _sc_stripe_body(uid_h, iid_h, eid_h, tu_h, ti_h, te_h,
                    pu_h, pi_h, pe_h, idx_v, t0_v, t1_v, t2_v, t3_v,
                    pack_v, s0, s1, s2, s3):
    wid = lax.axis_index("s") * NC + lax.axis_index("c")
    base = wid * (B // NW)
    lanes = lax.broadcasted_iota(i32, (NL,), 0)
    bufs = (t0_v, t1_v, t2_v, t3_v)
    sems = (s0, s1, s2, s3)
    for ids_h, t_h, o_h in ((uid_h, tu_h, pu_h), (iid_h, ti_h, pi_h),
                            (eid_h, te_h, pe_h)):
        pltpu.sync_copy(ids_h.at[pl.ds(base, CHUNK)], idx_v.at[pl.ds(0, CHUNK)])

        def issue(k, buf, sem):
            i = idx_v[pl.ds(k, NL)][0]
            col0 = pl.multiple_of((i >> 7) << 7, D2)
            pltpu.async_copy(t_h.at[:, pl.ds(col0, D2)], buf, sem)

        for b in range(NBUF):
            issue(b, bufs[b], sems[b])

        def grp(g, _):
            for b in range(NBUF):
                k = g * NBUF + b
                pltpu.make_async_copy(t_h.at[:, pl.ds(0, D2)], bufs[b],
                                      sems[b]).wait()
                i = idx_v[pl.ds(k, NL)][0]
                col0 = pl.multiple_of((i >> 7) << 7, D2)
                ii = jnp.broadcast_to(i - col0, (NL,))
                p = k >> 1
                off = (k & 1) * D
                for g2 in range(D // NL):
                    vals = plsc.load_gather(bufs[b], [lanes + g2 * NL, ii])
                    pack_v[p, pl.ds(off + g2 * NL, NL)] = vals

                @pl.when(k + NBUF < CHUNK)
                def _():
                    issue(k + NBUF, bufs[b], sems[b])
            return 0

        lax.fori_loop(0, CHUNK // NBUF, grp, 0)
        pltpu.sync_copy(pack_v, o_h.at[pl.ds(wid * (CHUNK // 2), CHUNK // 2)])


def _sc_stripe(uid, iid, eid, tu, ti, te):
    out_type = (
        jax.ShapeDtypeStruct((B // 2, D2), f32),
        jax.ShapeDtypeStruct((B // 2, D2), f32),
        jax.ShapeDtypeStruct((B // 2, D2), f32),
    )
    scratch_types = [
        pltpu.VMEM((CHUNK + NL,), i32),
        pltpu.VMEM((D, D2), f32),
        pltpu.VMEM((D, D2), f32),
        pltpu.VMEM((D, D2), f32),
        pltpu.VMEM((D, D2), f32),
        pltpu.VMEM((CHUNK // 2, D2), f32),
        pltpu.SemaphoreType.DMA,
        pltpu.SemaphoreType.DMA,
        pltpu.SemaphoreType.DMA,
        pltpu.SemaphoreType.DMA,
    ]
    fn = pl.kernel(_sc_stripe_body, out_type=out_type, mesh=_sc_mesh(),
                   scratch_types=scratch_types,
                   compiler_params=pltpu.CompilerParams(
                       needs_layout_passes=False,
                       use_tc_tiling_on_sc=True))
    return fn(uid, iid, eid, tu, ti, te)


def _sc_pair_body(p0_h, p1_h, pu_h, pi_h, pe_h, u2_h, i2_h, e2_h,
                  idx_v, idxs_v, rows_v, sem):
    wid = lax.axis_index("s") * NC + lax.axis_index("c")
    for p_h, src_h, out_h in ((p0_h, pu_h, u2_h), (p0_h, pi_h, i2_h),
                              (p1_h, pe_h, e2_h)):
        for c in range(NPAIR // NW // CHUNK):
            base = wid * (NPAIR // NW) + c * CHUNK
            pltpu.sync_copy(p_h.at[pl.ds(base, CHUNK)], idx_v)
            for g in range(CHUNK // NL):
                idxs_v[pl.ds(g * NL, NL)] = idx_v[pl.ds(g * NL, NL)] >> 1
            pltpu.async_copy(src_h.at[idxs_v], rows_v, sem).wait()
            pltpu.sync_copy(rows_v, out_h.at[pl.ds(base, CHUNK)])


def _sc_pair(p0, p1, pu, pi_, pe):
    out_type = (
        jax.ShapeDtypeStruct((NPAIR, D2), f32),
        jax.ShapeDtypeStruct((NPAIR, D2), f32),
        jax.ShapeDtypeStruct((NPAIR, D2), f32),
    )
    scratch_types = [
        pltpu.VMEM((CHUNK,), i32),
        pltpu.VMEM((CHUNK,), i32),
        pltpu.VMEM((CHUNK, D2), f32),
        pltpu.SemaphoreType.DMA,
    ]
    fn = pl.kernel(_sc_pair_body, out_type=out_type, mesh=_sc_mesh(),
                   scratch_types=scratch_types,
                   compiler_params=pltpu.CompilerParams(
                       needs_layout_passes=False,
                       use_tc_tiling_on_sc=True))
    return fn(p0, p1, pu, pi_, pe)


def _halfsel(packed, par):
    return jnp.where((par & 1) == 1, packed[:, D:], packed[:, :D])


def _prep_body(u_ref, i_ref, e_ref, u2_ref, i2_ref, e2_ref,
               p0c_ref, p1c_ref, ui_ref, en_ref, zd_ref, zp_ref):
    mix = MU * u_ref[...] + (1.0 - MU) * i_ref[...]
    nm = jnp.sqrt(jnp.sum(mix * mix, axis=1, keepdims=True))
    ui = mix / jnp.maximum(nm, EPS)
    ex = e_ref[...]
    ne = jnp.sqrt(jnp.sum(ex * ex, axis=1, keepdims=True))
    en = ex / jnp.maximum(ne, EPS)
    ui_ref[...] = ui
    en_ref[...] = en
    zd_ref[...] = jnp.sum(ui * en, axis=1, keepdims=True) / TEMP

    u2 = _halfsel(u2_ref[...], p0c_ref[...])
    i2 = _halfsel(i2_ref[...], p0c_ref[...])
    e2 = _halfsel(e2_ref[...], p1c_ref[...])
    mix2 = MU * u2 + (1.0 - MU) * i2
    n1 = jnp.maximum(jnp.sqrt(jnp.sum(mix2 * mix2, axis=1, keepdims=True)), EPS)
    n2 = jnp.maximum(jnp.sqrt(jnp.sum(e2 * e2, axis=1, keepdims=True)), EPS)
    dt = jnp.sum(mix2 * e2, axis=1, keepdims=True)
    zp_ref[...] = dt / (n1 * n2) / TEMP


def _tc_prep(u_rows, i_rows, e_rows, u2, i2, e2, p0c, p1c):
    G = 8
    RB = B // G
    PB = NPAIR // G
    return pl.pallas_call(
        _prep_body,
        grid=(G,),
        in_specs=[
            pl.BlockSpec((RB, D), lambda g: (g, 0)),
            pl.BlockSpec((RB, D), lambda g: (g, 0)),
            pl.BlockSpec((RB, D), lambda g: (g, 0)),
            pl.BlockSpec((PB, D2), lambda g: (g, 0)),
            pl.BlockSpec((PB, D2), lambda g: (g, 0)),
            pl.BlockSpec((PB, D2), lambda g: (g, 0)),
            pl.BlockSpec((PB, 1), lambda g: (g, 0)),
            pl.BlockSpec((PB, 1), lambda g: (g, 0)),
        ],
        out_specs=[
            pl.BlockSpec((RB, D), lambda g: (g, 0)),
            pl.BlockSpec((RB, D), lambda g: (g, 0)),
            pl.BlockSpec((RB, 1), lambda g: (g, 0)),
            pl.BlockSpec((PB, 1), lambda g: (g, 0)),
        ],
        out_shape=[
            jax.ShapeDtypeStruct((B, D), f32),
            jax.ShapeDtypeStruct((B, D), f32),
            jax.ShapeDtypeStruct((B, 1), f32),
            jax.ShapeDtypeStruct((NPAIR, 1), f32),
        ],
    )(u_rows, i_rows, e_rows, u2, i2, e2, p0c, p1c)


_ZT = 512


def _zpass_body(ui_ref, en_ref, rse_ref, rsz_ref, cse_ref):
    i = pl.program_id(0)
    j = pl.program_id(1)
    zt = lax.dot_general(ui_ref[...].astype(jnp.bfloat16),
                         en_ref[...].astype(jnp.bfloat16),
                         (((1,), (1,)), ((), ())),
                         preferred_element_type=f32) * (1.0 / TEMP)
    ez = jnp.exp(zt)
    rse_part = jnp.sum(ez, axis=1, keepdims=True)
    rsz_part = jnp.sum(zt, axis=1, keepdims=True)
    cse_part = jnp.sum(ez, axis=0, keepdims=True)

    @pl.when(j == 0)
    def _():
        rse_ref[...] = rse_part
        rsz_ref[...] = rsz_part

    @pl.when(j != 0)
    def _():
        rse_ref[...] += rse_part
        rsz_ref[...] += rsz_part

    @pl.when(i == 0)
    def _():
        cse_ref[:, pl.ds(j * _ZT, _ZT)] = cse_part

    @pl.when(i != 0)
    def _():
        cse_ref[:, pl.ds(j * _ZT, _ZT)] += cse_part


def _tc_zpass(ui_n, e_n):
    G = B // _ZT
    return pl.pallas_call(
        _zpass_body,
        grid=(G, G),
        in_specs=[
            pl.BlockSpec((_ZT, D), lambda i, j: (i, 0)),
            pl.BlockSpec((_ZT, D), lambda i, j: (j, 0)),
        ],
        out_specs=[
            pl.BlockSpec((_ZT, 1), lambda i, j: (i, 0)),
            pl.BlockSpec((_ZT, 1), lambda i, j: (i, 0)),
            pl.BlockSpec((1, B), lambda i, j: (0, 0)),
        ],
        out_shape=[
            jax.ShapeDtypeStruct((B, 1), f32),
            jax.ShapeDtypeStruct((B, 1), f32),
            jax.ShapeDtypeStruct((1, B), f32),
        ],
        compiler_params=pltpu.CompilerParams(
            dimension_semantics=("arbitrary", "arbitrary")),
    )(ui_n, e_n)


_PT = 1024
_CT = 512


def _cnt_body(p0c_ref, p1c_ref, p0r_ref, p1r_ref, cnt_ref):
    j = pl.program_id(1)
    p0c = p0c_ref[...]
    p1c = p1c_ref[...]
    keyc = p0c * B + p1c
    keyr = p0r_ref[...] * B + p1r_ref[...]
    eq = jnp.where(keyc == keyr, 1.0, 0.0)
    cnt_part = jnp.sum(eq, axis=1, keepdims=True)

    @pl.when(j == 0)
    def _():
        cnt_ref[...] = cnt_part + jnp.where(p0c == p1c, 1.0, 0.0)

    @pl.when(j != 0)
    def _():
        cnt_ref[...] += cnt_part


def _tc_cnt(p0c, p1c, p0r, p1r):
    G = NPAIR // _PT
    return pl.pallas_call(
        _cnt_body,
        grid=(G, G),
        in_specs=[
            pl.BlockSpec((_PT, 1), lambda i, j: (i, 0)),
            pl.BlockSpec((_PT, 1), lambda i, j: (i, 0)),
            pl.BlockSpec((1, _PT), lambda i, j: (0, j)),
            pl.BlockSpec((1, _PT), lambda i, j: (0, j)),
        ],
        out_specs=pl.BlockSpec((_PT, 1), lambda i, j: (i, 0)),
        out_shape=jax.ShapeDtypeStruct((NPAIR, 1), f32),
        compiler_params=pltpu.CompilerParams(
            dimension_semantics=("arbitrary", "arbitrary")),
    )(p0c, p1c, p0r, p1r)


def _pairs_body(p0c_ref, p1c_ref, zp_ref, cnt_ref, cse_ref,
                lsep_ref, lsec_ref, w3_ref):
    i = pl.program_id(0)
    j = pl.program_id(1)
    nj = pl.num_programs(1)
    p0c = p0c_ref[...]
    p1c = p1c_ref[...]
    lse_g = jnp.log(cse_ref[...])

    cols_g = lax.broadcasted_iota(i32, (1, _PT), 1) + j * _PT
    g_part = jnp.sum(jnp.where(p1c == cols_g, lse_g, 0.0),
                     axis=1, keepdims=True)

    @pl.when(j == 0)
    def _():
        lsep_ref[...] = g_part

    @pl.when(j != 0)
    def _():
        lsep_ref[...] += g_part

    rows_t = lax.broadcasted_iota(i32, (_CT, 1), 0) + i * _CT
    t_part = jnp.sum(jnp.where(rows_t == cols_g, lse_g, 0.0),
                     axis=1, keepdims=True)

    @pl.when(j == 0)
    def _():
        lsec_ref[...] = t_part

    @pl.when(j != 0)
    def _():
        lsec_ref[...] += t_part

    @pl.when(j == nj - 1)
    def _():
        w = 1.0 / cnt_ref[...]
        v = w * (2.0 * zp_ref[...] - lsep_ref[...])
        d = jnp.where(p0c == p1c, 1.0, 0.0)
        w3_ref[...] = jnp.concatenate([w, v, d, jnp.zeros_like(w)], axis=1)


def _tc_pairs(p0c, p1c, z_p, cnt, c_se):
    GI = NPAIR // _PT
    GJ = B // _PT
    return pl.pallas_call(
        _pairs_body,
        grid=(GI, GJ),
        in_specs=[
            pl.BlockSpec((_PT, 1), lambda i, j: (i, 0)),
            pl.BlockSpec((_PT, 1), lambda i, j: (i, 0)),
            pl.BlockSpec((_PT, 1), lambda i, j: (i, 0)),
            pl.BlockSpec((_PT, 1), lambda i, j: (i, 0)),
            pl.BlockSpec((1, _PT), lambda i, j: (0, j)),
        ],
        out_specs=[
            pl.BlockSpec((_PT, 1), lambda i, j: (i, 0)),
            pl.BlockSpec((_CT, 1), lambda i, j: (i, 0)),
            pl.BlockSpec((_PT, 4), lambda i, j: (i, 0)),
        ],
        out_shape=[
            jax.ShapeDtypeStruct((NPAIR, 1), f32),
            jax.ShapeDtypeStruct((B, 1), f32),
            jax.ShapeDtypeStruct((NPAIR, 4), f32),
        ],
        compiler_params=pltpu.CompilerParams(
            dimension_semantics=("arbitrary", "arbitrary")),
    )(p0c, p1c, z_p, cnt, c_se)


_AT = 1024


def _agg_body(p0r_ref, w3_ref, agg_ref):
    t = pl.program_id(0)
    k = pl.program_id(1)
    bf16 = jnp.bfloat16
    rows = lax.broadcasted_iota(i32, (_AT, 1), 0) + t * _AT
    oht = jnp.where(rows == p0r_ref[...], 1.0, 0.0).astype(bf16)
    part = lax.dot_general(oht, w3_ref[...].astype(bf16),
                           (((1,), (0,)), ((), ())),
                           preferred_element_type=f32)

    @pl.when(k == 0)
    def _():
        agg_ref[...] = part

    @pl.when(k != 0)
    def _():
        agg_ref[...] += part


def _tc_agg(p0r, w3):
    GT = B // _AT
    GK = NPAIR // _PT
    return pl.pallas_call(
        _agg_body,
        grid=(GT, GK),
        in_specs=[
            pl.BlockSpec((1, _PT), lambda t, k: (0, k)),
            pl.BlockSpec((_PT, 4), lambda t, k: (k, 0)),
        ],
        out_specs=pl.BlockSpec((_AT, 4), lambda t, k: (t, 0)),
        out_shape=jax.ShapeDtypeStruct((B, 4), f32),
        compiler_params=pltpu.CompilerParams(
            dimension_semantics=("arbitrary", "arbitrary")),
    )(p0r, w3)


def _final_body(a_ref, bv_ref, m_ref, rse_ref, rsz_ref, zd_ref, lsec_ref,
                out_ref):
    a = a_ref[...]
    bv = bv_ref[...]
    m = m_ref[...]
    lse_row = jnp.log(rse_ref[...])
    rs_z = rsz_ref[...]
    z_diag = zd_ref[...]
    lse_col = lsec_ref[...]
    s_col = jnp.sum(lse_col)

    w_d = 1.0 / (1.0 + m)
    num_pos = a + w_d
    loss_pos = bv - a * lse_row + w_d * (2.0 * z_diag - lse_row - lse_col)
    rowsum_ls = 2.0 * rs_z - float(B) * lse_row - s_col
    loss_neg = rowsum_ls - loss_pos
    num_neg = float(B) - num_pos
    loss = -jnp.sum(loss_pos / num_pos + loss_neg / num_neg) / float(B)
    out_ref[...] = loss * jnp.ones((1, 1), f32)


def _tc_final(agg, r_se, rs_z, z_diag, lse_col_c):
    lane = (32, 128)
    args = [
        agg[:, 0:1].reshape(lane), agg[:, 1:2].reshape(lane),
        agg[:, 2:3].reshape(lane), r_se.reshape(lane), rs_z.reshape(lane),
        z_diag.reshape(lane), lse_col_c.reshape(lane),
    ]
    return pl.pallas_call(
        _final_body,
        out_shape=jax.ShapeDtypeStruct((1, 1), f32),
    )(*args)


def kernel(user_ids, item_ids, exp_ids, pos_indices, user_table, item_table,
           exp_table):
    uid = user_ids.astype(i32)
    iid = item_ids.astype(i32)
    eid = exp_ids.astype(i32)
    p0 = pos_indices[:, 0].astype(i32)
    p1 = pos_indices[:, 1].astype(i32)

    p0c = p0.reshape(NPAIR, 1)
    p1c = p1.reshape(NPAIR, 1)
    p0r0 = p0.reshape(1, NPAIR)
    p1r0 = p1.reshape(1, NPAIR)
    cnt = _tc_cnt(p0c, p1c, p0r0, p1r0)

    pu, pi_, pe = _sc_stripe(uid, iid, eid, user_table.T, item_table.T,
                             exp_table.T)
    u2, i2, e2 = _sc_pair(p0, p1, pu, pi_, pe)
    ui_n, e_n, z_diag, z_p = _tc_prep(
        pu.reshape(B, D), pi_.reshape(B, D), pe.reshape(B, D),
        u2, i2, e2, p0c, p1c)
    r_se, rs_z, c_se = _tc_zpass(ui_n, e_n)

    lse_p, lse_col_c, w3 = _tc_pairs(p0c, p1c, z_p, cnt, c_se)
    agg = _tc_agg(p0r0, w3)
    out = _tc_final(agg, r_se, rs_z, z_diag, lse_col_c)
    return out[0, 0]


# speedup vs baseline: 3.6692x; 1.0775x over previous
import jax
import jax.numpy as jnp
from jax import lax
from jax.experimental import pallas as pl
from jax.experimental.pallas import tpu as pltpu
from jax.experimental.pallas import tpu_sc as plsc

MU = 0.5
TEMP = 0.1
B = 4096
D = 64
NPAIR = 8192
EPS = 1e-12

NC = 2
NS = 16
NL = 16
NW = NC * NS
CHUNK = 128

f32 = jnp.float32
i32 = jnp.int32


D2 = 2 * D


def _sc_mesh():
    return plsc.VectorSubcoreMesh(core_axis_name="c", subcore_axis_name="s",
                                  num_cores=NC, num_subcores=NS)


NBUF = 8


def _sc_stripe_body(uid_h, iid_h, eid_h, tu_h, ti_h, te_h,
                    pu_h, pi_h, pe_h, idx_v, t0_v, t1_v, t2_v, t3_v,
                    t4_v, t5_v, t6_v, t7_v,
                    pack_v, s0, s1, s2, s3, s4, s5, s6, s7):
    wid = lax.axis_index("s") * NC + lax.axis_index("c")
    base = wid * (B // NW)
    lanes = lax.broadcasted_iota(i32, (NL,), 0)
    bufs = (t0_v, t1_v, t2_v, t3_v, t4_v, t5_v, t6_v, t7_v)
    sems = (s0, s1, s2, s3, s4, s5, s6, s7)
    for ids_h, t_h, o_h in ((uid_h, tu_h, pu_h), (iid_h, ti_h, pi_h),
                            (eid_h, te_h, pe_h)):
        pltpu.sync_copy(ids_h.at[pl.ds(base, CHUNK)], idx_v.at[pl.ds(0, CHUNK)])

        def issue(k, buf, sem):
            i = idx_v[pl.ds(k, NL)][0]
            col0 = pl.multiple_of((i >> 7) << 7, D2)
            pltpu.async_copy(t_h.at[:, pl.ds(col0, D2)], buf, sem)

        for b in range(NBUF):
            issue(b, bufs[b], sems[b])

        def grp(g, _):
            for b in range(NBUF):
                k = g * NBUF + b
                pltpu.make_async_copy(t_h.at[:, pl.ds(0, D2)], bufs[b],
                                      sems[b]).wait()
                i = idx_v[pl.ds(k, NL)][0]
                col0 = pl.multiple_of((i >> 7) << 7, D2)
                ii = jnp.broadcast_to(i - col0, (NL,))
                p = k >> 1
                off = (k & 1) * D
                for g2 in range(D // NL):
                    vals = plsc.load_gather(bufs[b], [lanes + g2 * NL, ii])
                    pack_v[p, pl.ds(off + g2 * NL, NL)] = vals

                @pl.when(k + NBUF < CHUNK)
                def _():
                    issue(k + NBUF, bufs[b], sems[b])
            return 0

        lax.fori_loop(0, CHUNK // NBUF, grp, 0)
        pltpu.sync_copy(pack_v, o_h.at[pl.ds(wid * (CHUNK // 2), CHUNK // 2)])


def _sc_stripe(uid, iid, eid, tu, ti, te):
    out_type = (
        jax.ShapeDtypeStruct((B // 2, D2), f32),
        jax.ShapeDtypeStruct((B // 2, D2), f32),
        jax.ShapeDtypeStruct((B // 2, D2), f32),
    )
    scratch_types = [
        pltpu.VMEM((CHUNK + NL,), i32),
        pltpu.VMEM((D, D2), f32),
        pltpu.VMEM((D, D2), f32),
        pltpu.VMEM((D, D2), f32),
        pltpu.VMEM((D, D2), f32),
        pltpu.VMEM((D, D2), f32),
        pltpu.VMEM((D, D2), f32),
        pltpu.VMEM((D, D2), f32),
        pltpu.VMEM((D, D2), f32),
        pltpu.VMEM((CHUNK // 2, D2), f32),
        pltpu.SemaphoreType.DMA,
        pltpu.SemaphoreType.DMA,
        pltpu.SemaphoreType.DMA,
        pltpu.SemaphoreType.DMA,
        pltpu.SemaphoreType.DMA,
        pltpu.SemaphoreType.DMA,
        pltpu.SemaphoreType.DMA,
        pltpu.SemaphoreType.DMA,
    ]
    fn = pl.kernel(_sc_stripe_body, out_type=out_type, mesh=_sc_mesh(),
                   scratch_types=scratch_types,
                   compiler_params=pltpu.CompilerParams(
                       needs_layout_passes=False,
                       use_tc_tiling_on_sc=True))
    return fn(uid, iid, eid, tu, ti, te)


def _sc_pair_body(p0_h, p1_h, pu_h, pi_h, pe_h, u2_h, i2_h, e2_h,
                  idx_v, idxs_v, rows_v, sem):
    wid = lax.axis_index("s") * NC + lax.axis_index("c")
    for p_h, src_h, out_h in ((p0_h, pu_h, u2_h), (p0_h, pi_h, i2_h),
                              (p1_h, pe_h, e2_h)):
        for c in range(NPAIR // NW // CHUNK):
            base = wid * (NPAIR // NW) + c * CHUNK
            pltpu.sync_copy(p_h.at[pl.ds(base, CHUNK)], idx_v)
            for g in range(CHUNK // NL):
                idxs_v[pl.ds(g * NL, NL)] = idx_v[pl.ds(g * NL, NL)] >> 1
            pltpu.async_copy(src_h.at[idxs_v], rows_v, sem).wait()
            pltpu.sync_copy(rows_v, out_h.at[pl.ds(base, CHUNK)])


def _sc_pair(p0, p1, pu, pi_, pe):
    out_type = (
        jax.ShapeDtypeStruct((NPAIR, D2), f32),
        jax.ShapeDtypeStruct((NPAIR, D2), f32),
        jax.ShapeDtypeStruct((NPAIR, D2), f32),
    )
    scratch_types = [
        pltpu.VMEM((CHUNK,), i32),
        pltpu.VMEM((CHUNK,), i32),
        pltpu.VMEM((CHUNK, D2), f32),
        pltpu.SemaphoreType.DMA,
    ]
    fn = pl.kernel(_sc_pair_body, out_type=out_type, mesh=_sc_mesh(),
                   scratch_types=scratch_types,
                   compiler_params=pltpu.CompilerParams(
                       needs_layout_passes=False,
                       use_tc_tiling_on_sc=True))
    return fn(p0, p1, pu, pi_, pe)


def _halfsel(packed, par):
    return jnp.where((par & 1) == 1, packed[:, D:], packed[:, :D])


def _prep_body(u_ref, i_ref, e_ref, u2_ref, i2_ref, e2_ref,
               p0c_ref, p1c_ref, ui_ref, en_ref, zd_ref, zp_ref):
    mix = MU * u_ref[...] + (1.0 - MU) * i_ref[...]
    nm = jnp.sqrt(jnp.sum(mix * mix, axis=1, keepdims=True))
    ui = mix / jnp.maximum(nm, EPS)
    ex = e_ref[...]
    ne = jnp.sqrt(jnp.sum(ex * ex, axis=1, keepdims=True))
    en = ex / jnp.maximum(ne, EPS)
    ui_ref[...] = ui
    en_ref[...] = en
    zd_ref[...] = jnp.sum(ui * en, axis=1, keepdims=True) / TEMP

    u2 = _halfsel(u2_ref[...], p0c_ref[...])
    i2 = _halfsel(i2_ref[...], p0c_ref[...])
    e2 = _halfsel(e2_ref[...], p1c_ref[...])
    mix2 = MU * u2 + (1.0 - MU) * i2
    n1 = jnp.maximum(jnp.sqrt(jnp.sum(mix2 * mix2, axis=1, keepdims=True)), EPS)
    n2 = jnp.maximum(jnp.sqrt(jnp.sum(e2 * e2, axis=1, keepdims=True)), EPS)
    dt = jnp.sum(mix2 * e2, axis=1, keepdims=True)
    zp_ref[...] = dt / (n1 * n2) / TEMP


def _tc_prep(u_rows, i_rows, e_rows, u2, i2, e2, p0c, p1c):
    G = 8
    RB = B // G
    PB = NPAIR // G
    return pl.pallas_call(
        _prep_body,
        grid=(G,),
        in_specs=[
            pl.BlockSpec((RB, D), lambda g: (g, 0)),
            pl.BlockSpec((RB, D), lambda g: (g, 0)),
            pl.BlockSpec((RB, D), lambda g: (g, 0)),
            pl.BlockSpec((PB, D2), lambda g: (g, 0)),
            pl.BlockSpec((PB, D2), lambda g: (g, 0)),
            pl.BlockSpec((PB, D2), lambda g: (g, 0)),
            pl.BlockSpec((PB, 1), lambda g: (g, 0)),
            pl.BlockSpec((PB, 1), lambda g: (g, 0)),
        ],
        out_specs=[
            pl.BlockSpec((RB, D), lambda g: (g, 0)),
            pl.BlockSpec((RB, D), lambda g: (g, 0)),
            pl.BlockSpec((RB, 1), lambda g: (g, 0)),
            pl.BlockSpec((PB, 1), lambda g: (g, 0)),
        ],
        out_shape=[
            jax.ShapeDtypeStruct((B, D), f32),
            jax.ShapeDtypeStruct((B, D), f32),
            jax.ShapeDtypeStruct((B, 1), f32),
            jax.ShapeDtypeStruct((NPAIR, 1), f32),
        ],
    )(u_rows, i_rows, e_rows, u2, i2, e2, p0c, p1c)


_ZT = 512


def _zpass_body(ui_ref, en_ref, rse_ref, rsz_ref, cse_ref):
    i = pl.program_id(0)
    j = pl.program_id(1)
    zt = lax.dot_general(ui_ref[...].astype(jnp.bfloat16),
                         en_ref[...].astype(jnp.bfloat16),
                         (((1,), (1,)), ((), ())),
                         preferred_element_type=f32) * (1.0 / TEMP)
    ez = jnp.exp(zt)
    rse_part = jnp.sum(ez, axis=1, keepdims=True)
    rsz_part = jnp.sum(zt, axis=1, keepdims=True)
    cse_part = jnp.sum(ez, axis=0, keepdims=True)

    @pl.when(j == 0)
    def _():
        rse_ref[...] = rse_part
        rsz_ref[...] = rsz_part

    @pl.when(j != 0)
    def _():
        rse_ref[...] += rse_part
        rsz_ref[...] += rsz_part

    @pl.when(i == 0)
    def _():
        cse_ref[:, pl.ds(j * _ZT, _ZT)] = cse_part

    @pl.when(i != 0)
    def _():
        cse_ref[:, pl.ds(j * _ZT, _ZT)] += cse_part


def _tc_zpass(ui_n, e_n):
    G = B // _ZT
    return pl.pallas_call(
        _zpass_body,
        grid=(G, G),
        in_specs=[
            pl.BlockSpec((_ZT, D), lambda i, j: (i, 0)),
            pl.BlockSpec((_ZT, D), lambda i, j: (j, 0)),
        ],
        out_specs=[
            pl.BlockSpec((_ZT, 1), lambda i, j: (i, 0)),
            pl.BlockSpec((_ZT, 1), lambda i, j: (i, 0)),
            pl.BlockSpec((1, B), lambda i, j: (0, 0)),
        ],
        out_shape=[
            jax.ShapeDtypeStruct((B, 1), f32),
            jax.ShapeDtypeStruct((B, 1), f32),
            jax.ShapeDtypeStruct((1, B), f32),
        ],
        compiler_params=pltpu.CompilerParams(
            dimension_semantics=("arbitrary", "arbitrary")),
    )(ui_n, e_n)


_PT = 1024
_CT = 512


def _cnt_body(p0c_ref, p1c_ref, p0r_ref, p1r_ref, cnt_ref):
    j = pl.program_id(1)
    p0c = p0c_ref[...]
    p1c = p1c_ref[...]
    keyc = p0c * B + p1c
    keyr = p0r_ref[...] * B + p1r_ref[...]
    eq = jnp.where(keyc == keyr, 1.0, 0.0)
    cnt_part = jnp.sum(eq, axis=1, keepdims=True)

    @pl.when(j == 0)
    def _():
        cnt_ref[...] = cnt_part + jnp.where(p0c == p1c, 1.0, 0.0)

    @pl.when(j != 0)
    def _():
        cnt_ref[...] += cnt_part


def _tc_cnt(p0c, p1c, p0r, p1r):
    G = NPAIR // _PT
    return pl.pallas_call(
        _cnt_body,
        grid=(G, G),
        in_specs=[
            pl.BlockSpec((_PT, 1), lambda i, j: (i, 0)),
            pl.BlockSpec((_PT, 1), lambda i, j: (i, 0)),
            pl.BlockSpec((1, _PT), lambda i, j: (0, j)),
            pl.BlockSpec((1, _PT), lambda i, j: (0, j)),
        ],
        out_specs=pl.BlockSpec((_PT, 1), lambda i, j: (i, 0)),
        out_shape=jax.ShapeDtypeStruct((NPAIR, 1), f32),
        compiler_params=pltpu.CompilerParams(
            dimension_semantics=("arbitrary", "arbitrary")),
    )(p0c, p1c, p0r, p1r)


def _pairs_body(p0c_ref, p1c_ref, zp_ref, cnt_ref, cse_ref,
                lsep_ref, lsec_ref, w3_ref):
    i = pl.program_id(0)
    j = pl.program_id(1)
    nj = pl.num_programs(1)
    p0c = p0c_ref[...]
    p1c = p1c_ref[...]
    lse_g = jnp.log(cse_ref[...])

    cols_g = lax.broadcasted_iota(i32, (1, _PT), 1) + j * _PT
    g_part = jnp.sum(jnp.where(p1c == cols_g, lse_g, 0.0),
                     axis=1, keepdims=True)

    @pl.when(j == 0)
    def _():
        lsep_ref[...] = g_part

    @pl.when(j != 0)
    def _():
        lsep_ref[...] += g_part

    rows_t = lax.broadcasted_iota(i32, (_CT, 1), 0) + i * _CT
    t_part = jnp.sum(jnp.where(rows_t == cols_g, lse_g, 0.0),
                     axis=1, keepdims=True)

    @pl.when(j == 0)
    def _():
        lsec_ref[...] = t_part

    @pl.when(j != 0)
    def _():
        lsec_ref[...] += t_part

    @pl.when(j == nj - 1)
    def _():
        w = 1.0 / cnt_ref[...]
        v = w * (2.0 * zp_ref[...] - lsep_ref[...])
        d = jnp.where(p0c == p1c, 1.0, 0.0)
        w3_ref[...] = jnp.concatenate([w, v, d, jnp.zeros_like(w)], axis=1)


def _tc_pairs(p0c, p1c, z_p, cnt, c_se):
    GI = NPAIR // _PT
    GJ = B // _PT
    return pl.pallas_call(
        _pairs_body,
        grid=(GI, GJ),
        in_specs=[
            pl.BlockSpec((_PT, 1), lambda i, j: (i, 0)),
            pl.BlockSpec((_PT, 1), lambda i, j: (i, 0)),
            pl.BlockSpec((_PT, 1), lambda i, j: (i, 0)),
            pl.BlockSpec((_PT, 1), lambda i, j: (i, 0)),
            pl.BlockSpec((1, _PT), lambda i, j: (0, j)),
        ],
        out_specs=[
            pl.BlockSpec((_PT, 1), lambda i, j: (i, 0)),
            pl.BlockSpec((_CT, 1), lambda i, j: (i, 0)),
            pl.BlockSpec((_PT, 4), lambda i, j: (i, 0)),
        ],
        out_shape=[
            jax.ShapeDtypeStruct((NPAIR, 1), f32),
            jax.ShapeDtypeStruct((B, 1), f32),
            jax.ShapeDtypeStruct((NPAIR, 4), f32),
        ],
        compiler_params=pltpu.CompilerParams(
            dimension_semantics=("arbitrary", "arbitrary")),
    )(p0c, p1c, z_p, cnt, c_se)


_AT = 1024


def _agg_body(p0r_ref, w3_ref, agg_ref):
    t = pl.program_id(0)
    k = pl.program_id(1)
    bf16 = jnp.bfloat16
    rows = lax.broadcasted_iota(i32, (_AT, 1), 0) + t * _AT
    oht = jnp.where(rows == p0r_ref[...], 1.0, 0.0).astype(bf16)
    part = lax.dot_general(oht, w3_ref[...].astype(bf16),
                           (((1,), (0,)), ((), ())),
                           preferred_element_type=f32)

    @pl.when(k == 0)
    def _():
        agg_ref[...] = part

    @pl.when(k != 0)
    def _():
        agg_ref[...] += part


def _tc_agg(p0r, w3):
    GT = B // _AT
    GK = NPAIR // _PT
    return pl.pallas_call(
        _agg_body,
        grid=(GT, GK),
        in_specs=[
            pl.BlockSpec((1, _PT), lambda t, k: (0, k)),
            pl.BlockSpec((_PT, 4), lambda t, k: (k, 0)),
        ],
        out_specs=pl.BlockSpec((_AT, 4), lambda t, k: (t, 0)),
        out_shape=jax.ShapeDtypeStruct((B, 4), f32),
        compiler_params=pltpu.CompilerParams(
            dimension_semantics=("arbitrary", "arbitrary")),
    )(p0r, w3)


def _final_body(a_ref, bv_ref, m_ref, rse_ref, rsz_ref, zd_ref, lsec_ref,
                out_ref):
    a = a_ref[...]
    bv = bv_ref[...]
    m = m_ref[...]
    lse_row = jnp.log(rse_ref[...])
    rs_z = rsz_ref[...]
    z_diag = zd_ref[...]
    lse_col = lsec_ref[...]
    s_col = jnp.sum(lse_col)

    w_d = 1.0 / (1.0 + m)
    num_pos = a + w_d
    loss_pos = bv - a * lse_row + w_d * (2.0 * z_diag - lse_row - lse_col)
    rowsum_ls = 2.0 * rs_z - float(B) * lse_row - s_col
    loss_neg = rowsum_ls - loss_pos
    num_neg = float(B) - num_pos
    loss = -jnp.sum(loss_pos / num_pos + loss_neg / num_neg) / float(B)
    out_ref[...] = loss * jnp.ones((1, 1), f32)


def _tc_final(agg, r_se, rs_z, z_diag, lse_col_c):
    lane = (32, 128)
    args = [
        agg[:, 0:1].reshape(lane), agg[:, 1:2].reshape(lane),
        agg[:, 2:3].reshape(lane), r_se.reshape(lane), rs_z.reshape(lane),
        z_diag.reshape(lane), lse_col_c.reshape(lane),
    ]
    return pl.pallas_call(
        _final_body,
        out_shape=jax.ShapeDtypeStruct((1, 1), f32),
    )(*args)


def kernel(user_ids, item_ids, exp_ids, pos_indices, user_table, item_table,
           exp_table):
    uid = user_ids.astype(i32)
    iid = item_ids.astype(i32)
    eid = exp_ids.astype(i32)
    p0 = pos_indices[:, 0].astype(i32)
    p1 = pos_indices[:, 1].astype(i32)

    p0c = p0.reshape(NPAIR, 1)
    p1c = p1.reshape(NPAIR, 1)
    p0r0 = p0.reshape(1, NPAIR)
    p1r0 = p1.reshape(1, NPAIR)
    cnt = _tc_cnt(p0c, p1c, p0r0, p1r0)

    pu, pi_, pe = _sc_stripe(uid, iid, eid, user_table.T, item_table.T,
                             exp_table.T)
    u2, i2, e2 = _sc_pair(p0, p1, pu, pi_, pe)
    ui_n, e_n, z_diag, z_p = _tc_prep(
        pu.reshape(B, D), pi_.reshape(B, D), pe.reshape(B, D),
        u2, i2, e2, p0c, p1c)
    r_se, rs_z, c_se = _tc_zpass(ui_n, e_n)

    lse_p, lse_col_c, w3 = _tc_pairs(p0c, p1c, z_p, cnt, c_se)
    agg = _tc_agg(p0r0, w3)
    out = _tc_final(agg, r_se, rs_z, z_diag, lse_col_c)
    return out[0, 0]
